# Initial kernel scaffold; baseline (speedup 1.0000x reference)
import jax
import jax.numpy as jnp
from jax.experimental import pallas as pl


def kernel(x, x_s, params, edge_index, batch):
    raise NotImplementedError("write your pallas kernel here")


# speedup vs baseline: 4.0711x; 4.0711x over previous
import functools

import jax
import jax.numpy as jnp
from jax import lax
from jax.experimental import pallas as pl
from jax.experimental.pallas import tpu as pltpu
from jax.experimental.pallas import tpu_sc as plsc

_N = 10000
_E = 320000
_G = 128
_TILES = 16
_K = 125
_CH = _E // _TILES // _K
_GC = 32
_NG = _CH // _GC
_RPT = 624
_RREM = _N - _TILES * _RPT


def _sc_agg_body(tab, src_i, dst_i, zeros, out, src_v, dst_v, rows_v,
                 agg_sh, sem):
    c = lax.axis_index("c")
    s = lax.axis_index("s")
    r0 = s * _RPT
    pltpu.sync_copy(zeros.at[pl.ds(r0, _RPT)], agg_sh.at[pl.ds(r0, _RPT)])

    @pl.when(s == 0)
    def _():
        pltpu.sync_copy(zeros.at[pl.ds(_TILES * _RPT, _RREM)],
                        agg_sh.at[pl.ds(_TILES * _RPT, _RREM)])

    plsc.subcore_barrier()

    @pl.loop(0, _NG)
    def _grp(g):
        pltpu.sync_copy(src_i.at[c, s, pl.ds(g * _GC, _GC)], src_v)
        pltpu.sync_copy(dst_i.at[s, pl.ds(g * _GC, _GC)], dst_v)

        @pl.loop(0, _GC)
        def _chunk(j):
            pltpu.async_copy(tab.at[src_v.at[j]], rows_v, sem).wait()
            pltpu.sync_copy(rows_v, agg_sh.at[dst_v.at[j]], add=True)

    plsc.subcore_barrier()

    pltpu.sync_copy(agg_sh.at[pl.ds(r0, _RPT)], out.at[c, pl.ds(r0, _RPT)])

    @pl.when(s == 0)
    def _():
        pltpu.sync_copy(agg_sh.at[pl.ds(_TILES * _RPT, _RREM)],
                        out.at[c, pl.ds(_TILES * _RPT, _RREM)])


@functools.lru_cache(maxsize=None)
def _sc_agg_call():
    mesh = plsc.VectorSubcoreMesh(core_axis_name="c", subcore_axis_name="s")
    return pl.kernel(
        _sc_agg_body,
        out_type=jax.ShapeDtypeStruct((2, _N, 128), jnp.float32),
        mesh=mesh,
        scratch_types=[
            pltpu.VMEM((_GC, _K), jnp.int32),
            pltpu.VMEM((_GC, _K), jnp.int32),
            pltpu.VMEM((_K, 128), jnp.float32),
            pltpu.VMEM_SHARED((_N, 128), jnp.float32),
            pltpu.SemaphoreType.DMA,
        ],
    )


def _l1_body(x_ref, a_ref, w1, b1, w2, b2, lo, hi):
    u = x_ref[...] + a_ref[...]
    t = jnp.maximum(
        jnp.dot(u, w1[...], preferred_element_type=jnp.float32) + b1[...], 0.0)
    o = jnp.maximum(
        jnp.dot(t, w2[...], preferred_element_type=jnp.float32) + b2[...], 0.0)
    lo[...] = o[:, :128]
    hi[...] = o[:, 128:]


def _layer1(x, a, w1, b1, w2, b2):
    br = 1000
    return pl.pallas_call(
        _l1_body,
        grid=(_N // br,),
        in_specs=[
            pl.BlockSpec((br, 128), lambda i: (i, 0)),
            pl.BlockSpec((br, 128), lambda i: (i, 0)),
            pl.BlockSpec((128, 256), lambda i: (0, 0)),
            pl.BlockSpec((1, 256), lambda i: (0, 0)),
            pl.BlockSpec((256, 256), lambda i: (0, 0)),
            pl.BlockSpec((1, 256), lambda i: (0, 0)),
        ],
        out_specs=[pl.BlockSpec((br, 128), lambda i: (i, 0))] * 2,
        out_shape=[jax.ShapeDtypeStruct((_N, 128), jnp.float32)] * 2,
    )(x, a, w1, b1.reshape(1, -1), w2, b2.reshape(1, -1))


def _l2_body(hlo_ref, hhi_ref, alo_ref, ahi_ref, w1, b1, w2, b2, lo, hi):
    u = jnp.concatenate(
        [hlo_ref[...] + alo_ref[...], hhi_ref[...] + ahi_ref[...]], axis=1)
    t = jnp.maximum(
        jnp.dot(u, w1[...], preferred_element_type=jnp.float32) + b1[...], 0.0)
    o = jnp.maximum(
        jnp.dot(t, w2[...], preferred_element_type=jnp.float32) + b2[...], 0.0)
    lo[...] = o[:, :128]
    hi[...] = o[:, 128:]


def _layer2(hlo, hhi, alo, ahi, w1, b1, w2, b2):
    br = 1000
    return pl.pallas_call(
        _l2_body,
        grid=(_N // br,),
        in_specs=[
            pl.BlockSpec((br, 128), lambda i: (i, 0)),
            pl.BlockSpec((br, 128), lambda i: (i, 0)),
            pl.BlockSpec((br, 128), lambda i: (i, 0)),
            pl.BlockSpec((br, 128), lambda i: (i, 0)),
            pl.BlockSpec((256, 256), lambda i: (0, 0)),
            pl.BlockSpec((1, 256), lambda i: (0, 0)),
            pl.BlockSpec((256, 256), lambda i: (0, 0)),
            pl.BlockSpec((1, 256), lambda i: (0, 0)),
        ],
        out_specs=[pl.BlockSpec((br, 128), lambda i: (i, 0))] * 2,
        out_shape=[jax.ShapeDtypeStruct((_N, 128), jnp.float32)] * 2,
    )(hlo, hhi, alo, ahi, w1, b1.reshape(1, -1), w2, b2.reshape(1, -1))


def _pool_body(batch_ref, nf_ref, ns_ref, gf_ref, gs_ref):
    i = pl.program_id(0)
    bm = batch_ref[0]
    gi = lax.broadcasted_iota(jnp.int32, (_G, 1), 0)
    onehot = (gi == bm).astype(jnp.float32)

    @pl.when(i == 0)
    def _():
        gf_ref[...] = jnp.zeros_like(gf_ref)
        gs_ref[...] = jnp.zeros_like(gs_ref)

    gf_ref[...] += jnp.dot(onehot, nf_ref[...],
                           preferred_element_type=jnp.float32)
    gs_ref[...] += jnp.dot(onehot, ns_ref[...],
                           preferred_element_type=jnp.float32)


def _pool(batch3, nf, ns):
    br = 1000
    return pl.pallas_call(
        _pool_body,
        grid=(_N // br,),
        in_specs=[
            pl.BlockSpec((1, 1, br), lambda i: (i, 0, 0)),
            pl.BlockSpec((br, 512), lambda i: (i, 0)),
            pl.BlockSpec((br, 512), lambda i: (i, 0)),
        ],
        out_specs=[pl.BlockSpec((_G, 512), lambda i: (0, 0))] * 2,
        out_shape=[jax.ShapeDtypeStruct((_G, 512), jnp.float32)] * 2,
    )(batch3, nf, ns)


def _mlp_body(u_ref, w1, b1, w2, b2, o_ref):
    t = jnp.maximum(
        jnp.dot(u_ref[...], w1[...], preferred_element_type=jnp.float32)
        + b1[...], 0.0)
    o_ref[...] = (jnp.dot(t, w2[...], preferred_element_type=jnp.float32)
                  + b2[...])


def _mlp_head(u, w1, b1, w2, b2, br):
    r, din = u.shape
    dmid = w1.shape[1]
    dout = w2.shape[1]
    return pl.pallas_call(
        _mlp_body,
        grid=(r // br,),
        in_specs=[
            pl.BlockSpec((br, din), lambda i: (i, 0)),
            pl.BlockSpec((din, dmid), lambda i: (0, 0)),
            pl.BlockSpec((1, dmid), lambda i: (0, 0)),
            pl.BlockSpec((dmid, dout), lambda i: (0, 0)),
            pl.BlockSpec((1, dout), lambda i: (0, 0)),
        ],
        out_specs=pl.BlockSpec((br, dout), lambda i: (i, 0)),
        out_shape=jax.ShapeDtypeStruct((r, dout), jnp.float32),
    )(u, w1, b1.reshape(1, -1), w2, b2.reshape(1, -1))


def kernel(x, x_s, params, edge_index, batch):
    src = edge_index[0].astype(jnp.int32).reshape(_TILES, _CH, _K)
    src2 = jnp.stack([src, src + _N])
    dst = edge_index[1].astype(jnp.int32).reshape(_TILES, _CH, _K)
    zeros = jnp.zeros((_N, 128), jnp.float32)
    batch3 = batch.astype(jnp.int32).reshape(_N // 1000, 1, 1000)
    agg = _sc_agg_call()

    a1 = agg(jnp.concatenate([x, x_s], axis=0), src2, dst, zeros)
    h1f_lo, h1f_hi = _layer1(x, a1[0], *params["ef"][0])
    h1s_lo, h1s_hi = _layer1(x_s, a1[1], *params["es"][0])

    a2f = agg(jnp.concatenate([h1f_lo, h1f_hi], axis=0), src2, dst, zeros)
    a2s = agg(jnp.concatenate([h1s_lo, h1s_hi], axis=0), src2, dst, zeros)
    h2f_lo, h2f_hi = _layer2(h1f_lo, h1f_hi, a2f[0], a2f[1], *params["ef"][1])
    h2s_lo, h2s_hi = _layer2(h1s_lo, h1s_hi, a2s[0], a2s[1], *params["es"][1])

    n_f_cat = jnp.concatenate([h1f_lo, h1f_hi, h2f_lo, h2f_hi], axis=1)
    n_s_cat = jnp.concatenate([h1s_lo, h1s_hi, h2s_lo, h2s_hi], axis=1)

    g_f_raw, g_s_raw = _pool(batch3, n_f_cat, n_s_cat)
    g_cat = jnp.concatenate([g_f_raw, g_s_raw], axis=1)

    b = _mlp_head(g_cat, *params["pb"], br=_G)
    g_f = _mlp_head(g_f_raw, *params["pfg"], br=_G)
    g_s = _mlp_head(g_s_raw, *params["psg"], br=_G)
    n_f = _mlp_head(n_f_cat, *params["pfn"], br=1000)
    n_s = _mlp_head(n_s_cat, *params["psn"], br=1000)
    return (b, g_f, g_s, n_f, n_s)


# speedup vs baseline: 6.2256x; 1.5292x over previous
import functools

import jax
import jax.numpy as jnp
from jax import lax
from jax.experimental import pallas as pl
from jax.experimental.pallas import tpu as pltpu
from jax.experimental.pallas import tpu_sc as plsc

_N = 10000
_E = 320000
_G = 128
_TILES = 16
_K = 125
_CH = _E // _TILES // _K
_GC = 32
_NG = _CH // _GC
_RPT = 624
_RREM = _N - _TILES * _RPT


def _sc_agg_body(tab, src_i, dst_i, zeros, out, src_v, dst_v, rows_a, rows_b,
                 agg_sh, sem_a, sem_b, sem_i):
    c = lax.axis_index("c")
    s = lax.axis_index("s")
    r0 = s * _RPT
    pltpu.sync_copy(zeros.at[pl.ds(r0, _RPT)], agg_sh.at[pl.ds(r0, _RPT)])

    @pl.when(s == 0)
    def _():
        pltpu.sync_copy(zeros.at[pl.ds(_TILES * _RPT, _RREM)],
                        agg_sh.at[pl.ds(_TILES * _RPT, _RREM)])

    pltpu.sync_copy(src_i.at[c, s, pl.ds(0, _GC)], src_v.at[0])
    pltpu.sync_copy(dst_i.at[s, pl.ds(0, _GC)], dst_v.at[0])
    plsc.subcore_barrier()
    pltpu.async_copy(src_i.at[c, s, pl.ds(_GC, _GC)], src_v.at[1], sem_i)
    pltpu.async_copy(dst_i.at[s, pl.ds(_GC, _GC)], dst_v.at[1], sem_i)
    pltpu.async_copy(tab.at[src_v.at[0, 0]], rows_a, sem_a)

    @pl.loop(0, _CH // 2)
    def _pair(p):
        ch0 = 2 * p
        ch1 = ch0 + 1
        ch2 = ch0 + 2
        pltpu.async_copy(tab.at[src_v.at[(ch1 // _GC) % 2, ch1 % _GC]],
                         rows_b, sem_b)
        pltpu.make_async_copy(tab.at[src_v.at[(ch0 // _GC) % 2, ch0 % _GC]],
                              rows_a, sem_a).wait()
        pltpu.sync_copy(rows_a,
                        agg_sh.at[dst_v.at[(ch0 // _GC) % 2, ch0 % _GC]],
                        add=True)

        new_grp = jnp.logical_and(ch2 % _GC == 0, ch2 < _CH)

        @pl.when(new_grp)
        def _():
            pltpu.make_async_copy(src_i.at[c, s, pl.ds(0, _GC)],
                                  src_v.at[0], sem_i).wait()
            pltpu.make_async_copy(dst_i.at[s, pl.ds(0, _GC)],
                                  dst_v.at[0], sem_i).wait()

        @pl.when(jnp.logical_and(ch2 % _GC == 0, ch2 + _GC < _CH))
        def _():
            g3 = ch2 // _GC + 1
            pltpu.async_copy(src_i.at[c, s, pl.ds(g3 * _GC, _GC)],
                             src_v.at[g3 % 2], sem_i)
            pltpu.async_copy(dst_i.at[s, pl.ds(g3 * _GC, _GC)],
                             dst_v.at[g3 % 2], sem_i)

        @pl.when(ch2 < _CH)
        def _():
            pltpu.async_copy(tab.at[src_v.at[(ch2 // _GC) % 2, ch2 % _GC]],
                             rows_a, sem_a)

        pltpu.make_async_copy(tab.at[src_v.at[(ch1 // _GC) % 2, ch1 % _GC]],
                              rows_b, sem_b).wait()
        pltpu.sync_copy(rows_b,
                        agg_sh.at[dst_v.at[(ch1 // _GC) % 2, ch1 % _GC]],
                        add=True)

    plsc.subcore_barrier()

    pltpu.sync_copy(agg_sh.at[pl.ds(r0, _RPT)], out.at[c, pl.ds(r0, _RPT)])

    @pl.when(s == 0)
    def _():
        pltpu.sync_copy(agg_sh.at[pl.ds(_TILES * _RPT, _RREM)],
                        out.at[c, pl.ds(_TILES * _RPT, _RREM)])


@functools.lru_cache(maxsize=None)
def _sc_agg_call():
    mesh = plsc.VectorSubcoreMesh(core_axis_name="c", subcore_axis_name="s")
    return pl.kernel(
        _sc_agg_body,
        out_type=jax.ShapeDtypeStruct((2, _N, 128), jnp.float32),
        mesh=mesh,
        scratch_types=[
            pltpu.VMEM((2, _GC, _K), jnp.int32),
            pltpu.VMEM((2, _GC, _K), jnp.int32),
            pltpu.VMEM((_K, 128), jnp.float32),
            pltpu.VMEM((_K, 128), jnp.float32),
            pltpu.VMEM_SHARED((_N, 128), jnp.float32),
            pltpu.SemaphoreType.DMA,
            pltpu.SemaphoreType.DMA,
            pltpu.SemaphoreType.DMA,
        ],
    )


def _l1_body(x_ref, a_ref, w1, b1, w2, b2, lo, hi):
    u = x_ref[...] + a_ref[...]
    t = jnp.maximum(
        jnp.dot(u, w1[...], preferred_element_type=jnp.float32) + b1[...], 0.0)
    o = jnp.maximum(
        jnp.dot(t, w2[...], preferred_element_type=jnp.float32) + b2[...], 0.0)
    lo[...] = o[:, :128]
    hi[...] = o[:, 128:]


def _layer1(x, a, w1, b1, w2, b2):
    br = 1000
    return pl.pallas_call(
        _l1_body,
        grid=(_N // br,),
        in_specs=[
            pl.BlockSpec((br, 128), lambda i: (i, 0)),
            pl.BlockSpec((br, 128), lambda i: (i, 0)),
            pl.BlockSpec((128, 256), lambda i: (0, 0)),
            pl.BlockSpec((1, 256), lambda i: (0, 0)),
            pl.BlockSpec((256, 256), lambda i: (0, 0)),
            pl.BlockSpec((1, 256), lambda i: (0, 0)),
        ],
        out_specs=[pl.BlockSpec((br, 128), lambda i: (i, 0))] * 2,
        out_shape=[jax.ShapeDtypeStruct((_N, 128), jnp.float32)] * 2,
    )(x, a, w1, b1.reshape(1, -1), w2, b2.reshape(1, -1))


def _l2_body(hlo_ref, hhi_ref, alo_ref, ahi_ref, w1, b1, w2, b2, lo, hi):
    u = jnp.concatenate(
        [hlo_ref[...] + alo_ref[...], hhi_ref[...] + ahi_ref[...]], axis=1)
    t = jnp.maximum(
        jnp.dot(u, w1[...], preferred_element_type=jnp.float32) + b1[...], 0.0)
    o = jnp.maximum(
        jnp.dot(t, w2[...], preferred_element_type=jnp.float32) + b2[...], 0.0)
    lo[...] = o[:, :128]
    hi[...] = o[:, 128:]


def _layer2(hlo, hhi, alo, ahi, w1, b1, w2, b2):
    br = 1000
    return pl.pallas_call(
        _l2_body,
        grid=(_N // br,),
        in_specs=[
            pl.BlockSpec((br, 128), lambda i: (i, 0)),
            pl.BlockSpec((br, 128), lambda i: (i, 0)),
            pl.BlockSpec((br, 128), lambda i: (i, 0)),
            pl.BlockSpec((br, 128), lambda i: (i, 0)),
            pl.BlockSpec((256, 256), lambda i: (0, 0)),
            pl.BlockSpec((1, 256), lambda i: (0, 0)),
            pl.BlockSpec((256, 256), lambda i: (0, 0)),
            pl.BlockSpec((1, 256), lambda i: (0, 0)),
        ],
        out_specs=[pl.BlockSpec((br, 128), lambda i: (i, 0))] * 2,
        out_shape=[jax.ShapeDtypeStruct((_N, 128), jnp.float32)] * 2,
    )(hlo, hhi, alo, ahi, w1, b1.reshape(1, -1), w2, b2.reshape(1, -1))


def _pool_body(batch_ref, nf_ref, ns_ref, gf_ref, gs_ref):
    i = pl.program_id(0)
    bm = batch_ref[0]
    gi = lax.broadcasted_iota(jnp.int32, (_G, 1), 0)
    onehot = (gi == bm).astype(jnp.float32)

    @pl.when(i == 0)
    def _():
        gf_ref[...] = jnp.zeros_like(gf_ref)
        gs_ref[...] = jnp.zeros_like(gs_ref)

    gf_ref[...] += jnp.dot(onehot, nf_ref[...],
                           preferred_element_type=jnp.float32)
    gs_ref[...] += jnp.dot(onehot, ns_ref[...],
                           preferred_element_type=jnp.float32)


def _pool(batch3, nf, ns):
    br = 1000
    return pl.pallas_call(
        _pool_body,
        grid=(_N // br,),
        in_specs=[
            pl.BlockSpec((1, 1, br), lambda i: (i, 0, 0)),
            pl.BlockSpec((br, 512), lambda i: (i, 0)),
            pl.BlockSpec((br, 512), lambda i: (i, 0)),
        ],
        out_specs=[pl.BlockSpec((_G, 512), lambda i: (0, 0))] * 2,
        out_shape=[jax.ShapeDtypeStruct((_G, 512), jnp.float32)] * 2,
    )(batch3, nf, ns)


def _mlp_body(u_ref, w1, b1, w2, b2, o_ref):
    t = jnp.maximum(
        jnp.dot(u_ref[...], w1[...], preferred_element_type=jnp.float32)
        + b1[...], 0.0)
    o_ref[...] = (jnp.dot(t, w2[...], preferred_element_type=jnp.float32)
                  + b2[...])


def _mlp_head(u, w1, b1, w2, b2, br):
    r, din = u.shape
    dmid = w1.shape[1]
    dout = w2.shape[1]
    return pl.pallas_call(
        _mlp_body,
        grid=(r // br,),
        in_specs=[
            pl.BlockSpec((br, din), lambda i: (i, 0)),
            pl.BlockSpec((din, dmid), lambda i: (0, 0)),
            pl.BlockSpec((1, dmid), lambda i: (0, 0)),
            pl.BlockSpec((dmid, dout), lambda i: (0, 0)),
            pl.BlockSpec((1, dout), lambda i: (0, 0)),
        ],
        out_specs=pl.BlockSpec((br, dout), lambda i: (i, 0)),
        out_shape=jax.ShapeDtypeStruct((r, dout), jnp.float32),
    )(u, w1, b1.reshape(1, -1), w2, b2.reshape(1, -1))


def kernel(x, x_s, params, edge_index, batch):
    src = edge_index[0].astype(jnp.int32).reshape(_TILES, _CH, _K)
    src2 = jnp.stack([src, src + _N])
    dst = edge_index[1].astype(jnp.int32).reshape(_TILES, _CH, _K)
    zeros = jnp.zeros((_N, 128), jnp.float32)
    batch3 = batch.astype(jnp.int32).reshape(_N // 1000, 1, 1000)
    agg = _sc_agg_call()

    a1 = agg(jnp.concatenate([x, x_s], axis=0), src2, dst, zeros)
    h1f_lo, h1f_hi = _layer1(x, a1[0], *params["ef"][0])
    h1s_lo, h1s_hi = _layer1(x_s, a1[1], *params["es"][0])

    a2f = agg(jnp.concatenate([h1f_lo, h1f_hi], axis=0), src2, dst, zeros)
    a2s = agg(jnp.concatenate([h1s_lo, h1s_hi], axis=0), src2, dst, zeros)
    h2f_lo, h2f_hi = _layer2(h1f_lo, h1f_hi, a2f[0], a2f[1], *params["ef"][1])
    h2s_lo, h2s_hi = _layer2(h1s_lo, h1s_hi, a2s[0], a2s[1], *params["es"][1])

    n_f_cat = jnp.concatenate([h1f_lo, h1f_hi, h2f_lo, h2f_hi], axis=1)
    n_s_cat = jnp.concatenate([h1s_lo, h1s_hi, h2s_lo, h2s_hi], axis=1)

    g_f_raw, g_s_raw = _pool(batch3, n_f_cat, n_s_cat)
    g_cat = jnp.concatenate([g_f_raw, g_s_raw], axis=1)

    b = _mlp_head(g_cat, *params["pb"], br=_G)
    g_f = _mlp_head(g_f_raw, *params["pfg"], br=_G)
    g_s = _mlp_head(g_s_raw, *params["psg"], br=_G)
    n_f = _mlp_head(n_f_cat, *params["pfn"], br=1000)
    n_s = _mlp_head(n_s_cat, *params["psn"], br=1000)
    return (b, g_f, g_s, n_f, n_s)


# speedup vs baseline: 6.2344x; 1.0014x over previous
import functools

import jax
import jax.numpy as jnp
from jax import lax
from jax.experimental import pallas as pl
from jax.experimental.pallas import tpu as pltpu
from jax.experimental.pallas import tpu_sc as plsc

_N = 10000
_E = 320000
_G = 128
_TILES = 16
_K = 125
_CH = _E // _TILES // _K
_GC = 32
_NG = _CH // _GC
_RPT = 624
_RREM = _N - _TILES * _RPT


def _sc_agg_body(tab, src_i, dst_i, zeros, out, src_v, dst_v, rows_a, rows_b,
                 agg_sh, sem_a, sem_b, sem_i):
    c = lax.axis_index("c")
    s = lax.axis_index("s")
    r0 = s * _RPT
    pltpu.sync_copy(zeros.at[pl.ds(r0, _RPT)], agg_sh.at[pl.ds(r0, _RPT)])

    @pl.when(s == 0)
    def _():
        pltpu.sync_copy(zeros.at[pl.ds(_TILES * _RPT, _RREM)],
                        agg_sh.at[pl.ds(_TILES * _RPT, _RREM)])

    pltpu.sync_copy(src_i.at[c, s, pl.ds(0, _GC)], src_v.at[0])
    pltpu.sync_copy(dst_i.at[s, pl.ds(0, _GC)], dst_v.at[0])
    plsc.subcore_barrier()
    pltpu.async_copy(src_i.at[c, s, pl.ds(_GC, _GC)], src_v.at[1], sem_i)
    pltpu.async_copy(dst_i.at[s, pl.ds(_GC, _GC)], dst_v.at[1], sem_i)
    pltpu.async_copy(tab.at[src_v.at[0, 0]], rows_a, sem_a)

    @pl.loop(0, _CH // 2)
    def _pair(p):
        ch0 = 2 * p
        ch1 = ch0 + 1
        ch2 = ch0 + 2
        pltpu.async_copy(tab.at[src_v.at[(ch1 // _GC) % 2, ch1 % _GC]],
                         rows_b, sem_b)
        pltpu.make_async_copy(tab.at[src_v.at[(ch0 // _GC) % 2, ch0 % _GC]],
                              rows_a, sem_a).wait()
        pltpu.sync_copy(rows_a,
                        agg_sh.at[dst_v.at[(ch0 // _GC) % 2, ch0 % _GC]],
                        add=True)

        new_grp = jnp.logical_and(ch2 % _GC == 0, ch2 < _CH)

        @pl.when(new_grp)
        def _():
            pltpu.make_async_copy(src_i.at[c, s, pl.ds(0, _GC)],
                                  src_v.at[0], sem_i).wait()
            pltpu.make_async_copy(dst_i.at[s, pl.ds(0, _GC)],
                                  dst_v.at[0], sem_i).wait()

        @pl.when(ch2 < _CH)
        def _():
            pltpu.async_copy(tab.at[src_v.at[(ch2 // _GC) % 2, ch2 % _GC]],
                             rows_a, sem_a)

        pltpu.make_async_copy(tab.at[src_v.at[(ch1 // _GC) % 2, ch1 % _GC]],
                              rows_b, sem_b).wait()
        pltpu.sync_copy(rows_b,
                        agg_sh.at[dst_v.at[(ch1 // _GC) % 2, ch1 % _GC]],
                        add=True)

        @pl.when(jnp.logical_and(ch2 % _GC == 0, ch2 + _GC < _CH))
        def _():
            g3 = ch2 // _GC + 1
            pltpu.async_copy(src_i.at[c, s, pl.ds(g3 * _GC, _GC)],
                             src_v.at[g3 % 2], sem_i)
            pltpu.async_copy(dst_i.at[s, pl.ds(g3 * _GC, _GC)],
                             dst_v.at[g3 % 2], sem_i)

    plsc.subcore_barrier()

    pltpu.sync_copy(agg_sh.at[pl.ds(r0, _RPT)], out.at[c, pl.ds(r0, _RPT)])

    @pl.when(s == 0)
    def _():
        pltpu.sync_copy(agg_sh.at[pl.ds(_TILES * _RPT, _RREM)],
                        out.at[c, pl.ds(_TILES * _RPT, _RREM)])


@functools.lru_cache(maxsize=None)
def _sc_agg_call():
    mesh = plsc.VectorSubcoreMesh(core_axis_name="c", subcore_axis_name="s")
    return pl.kernel(
        _sc_agg_body,
        out_type=jax.ShapeDtypeStruct((2, _N, 128), jnp.float32),
        mesh=mesh,
        scratch_types=[
            pltpu.VMEM((2, _GC, _K), jnp.int32),
            pltpu.VMEM((2, _GC, _K), jnp.int32),
            pltpu.VMEM((_K, 128), jnp.float32),
            pltpu.VMEM((_K, 128), jnp.float32),
            pltpu.VMEM_SHARED((_N, 128), jnp.float32),
            pltpu.SemaphoreType.DMA,
            pltpu.SemaphoreType.DMA,
            pltpu.SemaphoreType.DMA,
        ],
    )


def _l1_body(x_ref, a_ref, w1, b1, w2, b2, lo, hi):
    u = x_ref[...] + a_ref[...]
    t = jnp.maximum(
        jnp.dot(u, w1[...], preferred_element_type=jnp.float32) + b1[...], 0.0)
    o = jnp.maximum(
        jnp.dot(t, w2[...], preferred_element_type=jnp.float32) + b2[...], 0.0)
    lo[...] = o[:, :128]
    hi[...] = o[:, 128:]


def _layer1(x, a, w1, b1, w2, b2):
    br = 1000
    return pl.pallas_call(
        _l1_body,
        grid=(_N // br,),
        in_specs=[
            pl.BlockSpec((br, 128), lambda i: (i, 0)),
            pl.BlockSpec((br, 128), lambda i: (i, 0)),
            pl.BlockSpec((128, 256), lambda i: (0, 0)),
            pl.BlockSpec((1, 256), lambda i: (0, 0)),
            pl.BlockSpec((256, 256), lambda i: (0, 0)),
            pl.BlockSpec((1, 256), lambda i: (0, 0)),
        ],
        out_specs=[pl.BlockSpec((br, 128), lambda i: (i, 0))] * 2,
        out_shape=[jax.ShapeDtypeStruct((_N, 128), jnp.float32)] * 2,
    )(x, a, w1, b1.reshape(1, -1), w2, b2.reshape(1, -1))


def _l2_body(hlo_ref, hhi_ref, alo_ref, ahi_ref, w1, b1, w2, b2, lo, hi):
    u = jnp.concatenate(
        [hlo_ref[...] + alo_ref[...], hhi_ref[...] + ahi_ref[...]], axis=1)
    t = jnp.maximum(
        jnp.dot(u, w1[...], preferred_element_type=jnp.float32) + b1[...], 0.0)
    o = jnp.maximum(
        jnp.dot(t, w2[...], preferred_element_type=jnp.float32) + b2[...], 0.0)
    lo[...] = o[:, :128]
    hi[...] = o[:, 128:]


def _layer2(hlo, hhi, alo, ahi, w1, b1, w2, b2):
    br = 1000
    return pl.pallas_call(
        _l2_body,
        grid=(_N // br,),
        in_specs=[
            pl.BlockSpec((br, 128), lambda i: (i, 0)),
            pl.BlockSpec((br, 128), lambda i: (i, 0)),
            pl.BlockSpec((br, 128), lambda i: (i, 0)),
            pl.BlockSpec((br, 128), lambda i: (i, 0)),
            pl.BlockSpec((256, 256), lambda i: (0, 0)),
            pl.BlockSpec((1, 256), lambda i: (0, 0)),
            pl.BlockSpec((256, 256), lambda i: (0, 0)),
            pl.BlockSpec((1, 256), lambda i: (0, 0)),
        ],
        out_specs=[pl.BlockSpec((br, 128), lambda i: (i, 0))] * 2,
        out_shape=[jax.ShapeDtypeStruct((_N, 128), jnp.float32)] * 2,
    )(hlo, hhi, alo, ahi, w1, b1.reshape(1, -1), w2, b2.reshape(1, -1))


def _pool_body(batch_ref, nf_ref, ns_ref, gf_ref, gs_ref):
    i = pl.program_id(0)
    bm = batch_ref[0]
    gi = lax.broadcasted_iota(jnp.int32, (_G, 1), 0)
    onehot = (gi == bm).astype(jnp.float32)

    @pl.when(i == 0)
    def _():
        gf_ref[...] = jnp.zeros_like(gf_ref)
        gs_ref[...] = jnp.zeros_like(gs_ref)

    gf_ref[...] += jnp.dot(onehot, nf_ref[...],
                           preferred_element_type=jnp.float32)
    gs_ref[...] += jnp.dot(onehot, ns_ref[...],
                           preferred_element_type=jnp.float32)


def _pool(batch3, nf, ns):
    br = 1000
    return pl.pallas_call(
        _pool_body,
        grid=(_N // br,),
        in_specs=[
            pl.BlockSpec((1, 1, br), lambda i: (i, 0, 0)),
            pl.BlockSpec((br, 512), lambda i: (i, 0)),
            pl.BlockSpec((br, 512), lambda i: (i, 0)),
        ],
        out_specs=[pl.BlockSpec((_G, 512), lambda i: (0, 0))] * 2,
        out_shape=[jax.ShapeDtypeStruct((_G, 512), jnp.float32)] * 2,
    )(batch3, nf, ns)


def _mlp_body(u_ref, w1, b1, w2, b2, o_ref):
    t = jnp.maximum(
        jnp.dot(u_ref[...], w1[...], preferred_element_type=jnp.float32)
        + b1[...], 0.0)
    o_ref[...] = (jnp.dot(t, w2[...], preferred_element_type=jnp.float32)
                  + b2[...])


def _mlp_head(u, w1, b1, w2, b2, br):
    r, din = u.shape
    dmid = w1.shape[1]
    dout = w2.shape[1]
    return pl.pallas_call(
        _mlp_body,
        grid=(r // br,),
        in_specs=[
            pl.BlockSpec((br, din), lambda i: (i, 0)),
            pl.BlockSpec((din, dmid), lambda i: (0, 0)),
            pl.BlockSpec((1, dmid), lambda i: (0, 0)),
            pl.BlockSpec((dmid, dout), lambda i: (0, 0)),
            pl.BlockSpec((1, dout), lambda i: (0, 0)),
        ],
        out_specs=pl.BlockSpec((br, dout), lambda i: (i, 0)),
        out_shape=jax.ShapeDtypeStruct((r, dout), jnp.float32),
    )(u, w1, b1.reshape(1, -1), w2, b2.reshape(1, -1))


def kernel(x, x_s, params, edge_index, batch):
    src = edge_index[0].astype(jnp.int32).reshape(_TILES, _CH, _K)
    src2 = jnp.stack([src, src + _N])
    dst = edge_index[1].astype(jnp.int32).reshape(_TILES, _CH, _K)
    zeros = jnp.zeros((_N, 128), jnp.float32)
    batch3 = batch.astype(jnp.int32).reshape(_N // 1000, 1, 1000)
    agg = _sc_agg_call()

    a1 = agg(jnp.concatenate([x, x_s], axis=0), src2, dst, zeros)
    h1f_lo, h1f_hi = _layer1(x, a1[0], *params["ef"][0])
    h1s_lo, h1s_hi = _layer1(x_s, a1[1], *params["es"][0])

    a2f = agg(jnp.concatenate([h1f_lo, h1f_hi], axis=0), src2, dst, zeros)
    a2s = agg(jnp.concatenate([h1s_lo, h1s_hi], axis=0), src2, dst, zeros)
    h2f_lo, h2f_hi = _layer2(h1f_lo, h1f_hi, a2f[0], a2f[1], *params["ef"][1])
    h2s_lo, h2s_hi = _layer2(h1s_lo, h1s_hi, a2s[0], a2s[1], *params["es"][1])

    n_f_cat = jnp.concatenate([h1f_lo, h1f_hi, h2f_lo, h2f_hi], axis=1)
    n_s_cat = jnp.concatenate([h1s_lo, h1s_hi, h2s_lo, h2s_hi], axis=1)

    g_f_raw, g_s_raw = _pool(batch3, n_f_cat, n_s_cat)
    g_cat = jnp.concatenate([g_f_raw, g_s_raw], axis=1)

    b = _mlp_head(g_cat, *params["pb"], br=_G)
    g_f = _mlp_head(g_f_raw, *params["pfg"], br=_G)
    g_s = _mlp_head(g_s_raw, *params["psg"], br=_G)
    n_f = _mlp_head(n_f_cat, *params["pfn"], br=1000)
    n_s = _mlp_head(n_s_cat, *params["psn"], br=1000)
    return (b, g_f, g_s, n_f, n_s)


# speedup vs baseline: 6.5205x; 1.0459x over previous
import functools

import jax
import jax.numpy as jnp
from jax import lax
from jax.experimental import pallas as pl
from jax.experimental.pallas import tpu as pltpu
from jax.experimental.pallas import tpu_sc as plsc

_N = 10000
_E = 320000
_G = 128
_TILES = 16
_K = 125
_CH = _E // _TILES // _K
_GC = 32
_NG = _CH // _GC
_RPT = 624
_RREM = _N - _TILES * _RPT


def _sc_agg_body(tab, src_i, dst_i, zeros, out, src_v, dst_v, rows_a, rows_b,
                 agg_sh, sem_a, sem_b, sem_i):
    c = lax.axis_index("c")
    s = lax.axis_index("s")
    r0 = s * _RPT
    pltpu.sync_copy(zeros.at[pl.ds(r0, _RPT)], agg_sh.at[pl.ds(r0, _RPT)])

    @pl.when(s == 0)
    def _():
        pltpu.sync_copy(zeros.at[pl.ds(_TILES * _RPT, _RREM)],
                        agg_sh.at[pl.ds(_TILES * _RPT, _RREM)])

    pltpu.sync_copy(src_i.at[c, s, pl.ds(0, _GC)], src_v.at[0])
    pltpu.sync_copy(dst_i.at[s, pl.ds(0, _GC)], dst_v.at[0])
    plsc.subcore_barrier()
    pltpu.async_copy(src_i.at[c, s, pl.ds(_GC, _GC)], src_v.at[1], sem_i)
    pltpu.async_copy(dst_i.at[s, pl.ds(_GC, _GC)], dst_v.at[1], sem_i)
    pltpu.async_copy(tab.at[src_v.at[0, 0]], rows_a, sem_a)

    @pl.loop(0, _CH // 2)
    def _pair(p):
        ch0 = 2 * p
        ch1 = ch0 + 1
        ch2 = ch0 + 2
        pltpu.async_copy(tab.at[src_v.at[(ch1 // _GC) % 2, ch1 % _GC]],
                         rows_b, sem_b)
        pltpu.make_async_copy(tab.at[src_v.at[(ch0 // _GC) % 2, ch0 % _GC]],
                              rows_a, sem_a).wait()
        pltpu.sync_copy(rows_a,
                        agg_sh.at[dst_v.at[(ch0 // _GC) % 2, ch0 % _GC]],
                        add=True)

        new_grp = jnp.logical_and(ch2 % _GC == 0, ch2 < _CH)

        @pl.when(new_grp)
        def _():
            pltpu.make_async_copy(src_i.at[c, s, pl.ds(0, _GC)],
                                  src_v.at[0], sem_i).wait()
            pltpu.make_async_copy(dst_i.at[s, pl.ds(0, _GC)],
                                  dst_v.at[0], sem_i).wait()

        @pl.when(ch2 < _CH)
        def _():
            pltpu.async_copy(tab.at[src_v.at[(ch2 // _GC) % 2, ch2 % _GC]],
                             rows_a, sem_a)

        pltpu.make_async_copy(tab.at[src_v.at[(ch1 // _GC) % 2, ch1 % _GC]],
                              rows_b, sem_b).wait()
        pltpu.sync_copy(rows_b,
                        agg_sh.at[dst_v.at[(ch1 // _GC) % 2, ch1 % _GC]],
                        add=True)

        @pl.when(jnp.logical_and(ch2 % _GC == 0, ch2 + _GC < _CH))
        def _():
            g3 = ch2 // _GC + 1
            pltpu.async_copy(src_i.at[c, s, pl.ds(g3 * _GC, _GC)],
                             src_v.at[g3 % 2], sem_i)
            pltpu.async_copy(dst_i.at[s, pl.ds(g3 * _GC, _GC)],
                             dst_v.at[g3 % 2], sem_i)

    plsc.subcore_barrier()

    pltpu.sync_copy(agg_sh.at[pl.ds(r0, _RPT)], out.at[c, pl.ds(r0, _RPT)])

    @pl.when(s == 0)
    def _():
        pltpu.sync_copy(agg_sh.at[pl.ds(_TILES * _RPT, _RREM)],
                        out.at[c, pl.ds(_TILES * _RPT, _RREM)])


@functools.lru_cache(maxsize=None)
def _sc_agg_call():
    mesh = plsc.VectorSubcoreMesh(core_axis_name="c", subcore_axis_name="s")
    return pl.kernel(
        _sc_agg_body,
        out_type=jax.ShapeDtypeStruct((2, _N, 128), jnp.float32),
        mesh=mesh,
        scratch_types=[
            pltpu.VMEM((2, _GC, _K), jnp.int32),
            pltpu.VMEM((2, _GC, _K), jnp.int32),
            pltpu.VMEM((_K, 128), jnp.float32),
            pltpu.VMEM((_K, 128), jnp.float32),
            pltpu.VMEM_SHARED((_N, 128), jnp.float32),
            pltpu.SemaphoreType.DMA,
            pltpu.SemaphoreType.DMA,
            pltpu.SemaphoreType.DMA,
        ],
    )


_BR = 1000


def _mlp_blk(u, w1, b1, w2, b2):
    t = jnp.maximum(
        jnp.dot(u, w1[...], preferred_element_type=jnp.float32) + b1[...], 0.0)
    return jnp.dot(t, w2[...], preferred_element_type=jnp.float32) + b2[...]


def _l1_body(x_ref, xs_ref, af_ref, as_ref, fw1, fb1, fw2, fb2,
             sw1, sb1, sw2, sb2, flo, fhi, slo, shi):
    of = jnp.maximum(_mlp_blk(x_ref[...] + af_ref[...], fw1, fb1, fw2, fb2),
                     0.0)
    flo[...] = of[:, :128]
    fhi[...] = of[:, 128:]
    os = jnp.maximum(_mlp_blk(xs_ref[...] + as_ref[...], sw1, sb1, sw2, sb2),
                     0.0)
    slo[...] = os[:, :128]
    shi[...] = os[:, 128:]


def _layer1(x, x_s, af, a_s, pf, ps):
    row = pl.BlockSpec((_BR, 128), lambda i: (i, 0))
    wspec = lambda shp: pl.BlockSpec(shp, lambda i: (0, 0))
    return pl.pallas_call(
        _l1_body,
        grid=(_N // _BR,),
        in_specs=[row, row, row, row,
                  wspec((128, 256)), wspec((1, 256)),
                  wspec((256, 256)), wspec((1, 256)),
                  wspec((128, 256)), wspec((1, 256)),
                  wspec((256, 256)), wspec((1, 256))],
        out_specs=[row] * 4,
        out_shape=[jax.ShapeDtypeStruct((_N, 128), jnp.float32)] * 4,
    )(x, x_s, af, a_s,
      pf[0], pf[1].reshape(1, -1), pf[2], pf[3].reshape(1, -1),
      ps[0], ps[1].reshape(1, -1), ps[2], ps[3].reshape(1, -1))


def _l2_body(flo_ref, fhi_ref, slo_ref, shi_ref, aflo, afhi, aslo, ashi,
             fw1, fb1, fw2, fb2, sw1, sb1, sw2, sb2,
             oflo, ofhi, oslo, oshi):
    uf = jnp.concatenate(
        [flo_ref[...] + aflo[...], fhi_ref[...] + afhi[...]], axis=1)
    of = jnp.maximum(_mlp_blk(uf, fw1, fb1, fw2, fb2), 0.0)
    oflo[...] = of[:, :128]
    ofhi[...] = of[:, 128:]
    us = jnp.concatenate(
        [slo_ref[...] + aslo[...], shi_ref[...] + ashi[...]], axis=1)
    os = jnp.maximum(_mlp_blk(us, sw1, sb1, sw2, sb2), 0.0)
    oslo[...] = os[:, :128]
    oshi[...] = os[:, 128:]


def _layer2(flo, fhi, slo, shi, aflo, afhi, aslo, ashi, pf, ps):
    row = pl.BlockSpec((_BR, 128), lambda i: (i, 0))
    wspec = lambda shp: pl.BlockSpec(shp, lambda i: (0, 0))
    return pl.pallas_call(
        _l2_body,
        grid=(_N // _BR,),
        in_specs=[row] * 8 + [
            wspec((256, 256)), wspec((1, 256)),
            wspec((256, 256)), wspec((1, 256)),
            wspec((256, 256)), wspec((1, 256)),
            wspec((256, 256)), wspec((1, 256))],
        out_specs=[row] * 4,
        out_shape=[jax.ShapeDtypeStruct((_N, 128), jnp.float32)] * 4,
    )(flo, fhi, slo, shi, aflo, afhi, aslo, ashi,
      pf[0], pf[1].reshape(1, -1), pf[2], pf[3].reshape(1, -1),
      ps[0], ps[1].reshape(1, -1), ps[2], ps[3].reshape(1, -1))


def _nheads_body(batch_ref, f1lo, f1hi, f2lo, f2hi, s1lo, s1hi, s2lo, s2hi,
                 fw1, fb1, fw2, fb2, sw1, sb1, sw2, sb2,
                 nf_ref, ns_ref, gf_ref, gs_ref):
    i = pl.program_id(0)
    ncf = jnp.concatenate(
        [f1lo[...], f1hi[...], f2lo[...], f2hi[...]], axis=1)
    ncs = jnp.concatenate(
        [s1lo[...], s1hi[...], s2lo[...], s2hi[...]], axis=1)
    nf_ref[...] = _mlp_blk(ncf, fw1, fb1, fw2, fb2)
    ns_ref[...] = _mlp_blk(ncs, sw1, sb1, sw2, sb2)

    bm = batch_ref[0]
    gi = lax.broadcasted_iota(jnp.int32, (_G, 1), 0)
    onehot = (gi == bm).astype(jnp.float32)

    @pl.when(i == 0)
    def _():
        gf_ref[...] = jnp.zeros_like(gf_ref)
        gs_ref[...] = jnp.zeros_like(gs_ref)

    gf_ref[...] += jnp.dot(onehot, ncf, preferred_element_type=jnp.float32)
    gs_ref[...] += jnp.dot(onehot, ncs, preferred_element_type=jnp.float32)


def _nheads(batch3, hf, hs, pf, ps):
    row = pl.BlockSpec((_BR, 128), lambda i: (i, 0))
    wspec = lambda shp: pl.BlockSpec(shp, lambda i: (0, 0))
    big = pl.BlockSpec((_BR, 512), lambda i: (i, 0))
    acc = pl.BlockSpec((_G, 512), lambda i: (0, 0))
    return pl.pallas_call(
        _nheads_body,
        grid=(_N // _BR,),
        in_specs=[pl.BlockSpec((1, 1, _BR), lambda i: (i, 0, 0))]
        + [row] * 8 + [
            wspec((512, 512)), wspec((1, 512)),
            wspec((512, 512)), wspec((1, 512)),
            wspec((512, 512)), wspec((1, 512)),
            wspec((512, 512)), wspec((1, 512))],
        out_specs=[big, big, acc, acc],
        out_shape=[jax.ShapeDtypeStruct((_N, 512), jnp.float32)] * 2
        + [jax.ShapeDtypeStruct((_G, 512), jnp.float32)] * 2,
    )(batch3, *hf, *hs,
      pf[0], pf[1].reshape(1, -1), pf[2], pf[3].reshape(1, -1),
      ps[0], ps[1].reshape(1, -1), ps[2], ps[3].reshape(1, -1))


def _gheads_body(gf_ref, gs_ref, bw1, bb1, bw2, bb2,
                 fw1, fb1, fw2, fb2, sw1, sb1, sw2, sb2,
                 b_ref, ogf_ref, ogs_ref):
    gcat = jnp.concatenate([gf_ref[...], gs_ref[...]], axis=1)
    b_ref[...] = _mlp_blk(gcat, bw1, bb1, bw2, bb2)
    ogf_ref[...] = _mlp_blk(gf_ref[...], fw1, fb1, fw2, fb2)
    ogs_ref[...] = _mlp_blk(gs_ref[...], sw1, sb1, sw2, sb2)


def _gheads(gf, gs, pb, pf, ps):
    wspec = lambda shp: pl.BlockSpec(shp, lambda i: (0, 0))
    g = pl.BlockSpec((_G, 512), lambda i: (0, 0))
    return pl.pallas_call(
        _gheads_body,
        grid=(1,),
        in_specs=[g, g,
                  wspec((1024, 512)), wspec((1, 512)),
                  wspec((512, 512)), wspec((1, 512)),
                  wspec((512, 512)), wspec((1, 512)),
                  wspec((512, 512)), wspec((1, 512)),
                  wspec((512, 512)), wspec((1, 512)),
                  wspec((512, 512)), wspec((1, 512))],
        out_specs=[g, g, g],
        out_shape=[jax.ShapeDtypeStruct((_G, 512), jnp.float32)] * 3,
    )(gf, gs,
      pb[0], pb[1].reshape(1, -1), pb[2], pb[3].reshape(1, -1),
      pf[0], pf[1].reshape(1, -1), pf[2], pf[3].reshape(1, -1),
      ps[0], ps[1].reshape(1, -1), ps[2], ps[3].reshape(1, -1))


def kernel(x, x_s, params, edge_index, batch):
    src = edge_index[0].astype(jnp.int32).reshape(_TILES, _CH, _K)
    src2 = jnp.stack([src, src + _N])
    dst = edge_index[1].astype(jnp.int32).reshape(_TILES, _CH, _K)
    zeros = jnp.zeros((_N, 128), jnp.float32)
    batch3 = batch.astype(jnp.int32).reshape(_N // 1000, 1, 1000)
    agg = _sc_agg_call()

    a1 = agg(jnp.concatenate([x, x_s], axis=0), src2, dst, zeros)
    h1f_lo, h1f_hi, h1s_lo, h1s_hi = _layer1(
        x, x_s, a1[0], a1[1], params["ef"][0], params["es"][0])

    a2f = agg(jnp.concatenate([h1f_lo, h1f_hi], axis=0), src2, dst, zeros)
    a2s = agg(jnp.concatenate([h1s_lo, h1s_hi], axis=0), src2, dst, zeros)
    h2f_lo, h2f_hi, h2s_lo, h2s_hi = _layer2(
        h1f_lo, h1f_hi, h1s_lo, h1s_hi, a2f[0], a2f[1], a2s[0], a2s[1],
        params["ef"][1], params["es"][1])

    n_f, n_s, g_f_raw, g_s_raw = _nheads(
        batch3, (h1f_lo, h1f_hi, h2f_lo, h2f_hi),
        (h1s_lo, h1s_hi, h2s_lo, h2s_hi), params["pfn"], params["psn"])
    b, g_f, g_s = _gheads(g_f_raw, g_s_raw,
                          params["pb"], params["pfg"], params["psg"])
    return (b, g_f, g_s, n_f, n_s)


# speedup vs baseline: 6.5327x; 1.0019x over previous
import functools

import jax
import jax.numpy as jnp
from jax import lax
from jax.experimental import pallas as pl
from jax.experimental.pallas import tpu as pltpu
from jax.experimental.pallas import tpu_sc as plsc

_N = 10000
_E = 320000
_G = 128
_TILES = 16
_K = 125
_CH = _E // _TILES // _K
_GC = 32
_NG = _CH // _GC
_RPT = 624
_RREM = _N - _TILES * _RPT


def _sc_agg_body(tab, src_i, dst_i, zeros, out, src_v, dst_v, rows_a, rows_b,
                 agg_sh, sem_a, sem_b, sem_i, sem_sa, sem_sb):
    c = lax.axis_index("c")
    s = lax.axis_index("s")
    r0 = s * _RPT
    pltpu.sync_copy(zeros.at[pl.ds(r0, _RPT)], agg_sh.at[pl.ds(r0, _RPT)])

    @pl.when(s == 0)
    def _():
        pltpu.sync_copy(zeros.at[pl.ds(_TILES * _RPT, _RREM)],
                        agg_sh.at[pl.ds(_TILES * _RPT, _RREM)])

    pltpu.sync_copy(src_i.at[c, s, pl.ds(0, _GC)], src_v.at[0])
    pltpu.sync_copy(dst_i.at[s, pl.ds(0, _GC)], dst_v.at[0])
    plsc.subcore_barrier()
    pltpu.async_copy(src_i.at[c, s, pl.ds(_GC, _GC)], src_v.at[1], sem_i)
    pltpu.async_copy(dst_i.at[s, pl.ds(_GC, _GC)], dst_v.at[1], sem_i)
    pltpu.async_copy(tab.at[src_v.at[0, 0]], rows_a, sem_a)
    pltpu.async_copy(tab.at[src_v.at[0, 1]], rows_b, sem_b)

    @pl.loop(0, _CH // 2)
    def _pair(p):
        ch0 = 2 * p
        ch1 = ch0 + 1
        ch2 = ch0 + 2
        ch3 = ch0 + 3
        s0 = (ch0 // _GC) % 2
        s2 = (ch2 // _GC) % 2

        pltpu.make_async_copy(tab.at[src_v.at[s0, ch0 % _GC]],
                              rows_a, sem_a).wait()
        pltpu.async_copy(rows_a, agg_sh.at[dst_v.at[s0, ch0 % _GC]],
                         sem_sa, add=True)

        @pl.when(jnp.logical_and(ch2 % _GC == 0, ch2 < _CH))
        def _():
            pltpu.make_async_copy(src_i.at[c, s, pl.ds(0, _GC)],
                                  src_v.at[0], sem_i).wait()
            pltpu.make_async_copy(dst_i.at[s, pl.ds(0, _GC)],
                                  dst_v.at[0], sem_i).wait()

        @pl.when(ch2 < _CH)
        def _():
            pltpu.make_async_copy(rows_a,
                                  agg_sh.at[dst_v.at[s0, ch0 % _GC]],
                                  sem_sa).wait()
            pltpu.async_copy(tab.at[src_v.at[s2, ch2 % _GC]], rows_a, sem_a)

        pltpu.make_async_copy(tab.at[src_v.at[s0, ch1 % _GC]],
                              rows_b, sem_b).wait()
        pltpu.async_copy(rows_b, agg_sh.at[dst_v.at[s0, ch1 % _GC]],
                         sem_sb, add=True)

        @pl.when(ch3 < _CH)
        def _():
            pltpu.make_async_copy(rows_b,
                                  agg_sh.at[dst_v.at[s0, ch1 % _GC]],
                                  sem_sb).wait()
            pltpu.async_copy(tab.at[src_v.at[s2, ch3 % _GC]], rows_b, sem_b)

        @pl.when(jnp.logical_and(ch2 % _GC == 0, ch2 + _GC < _CH))
        def _():
            g3 = ch2 // _GC + 1
            pltpu.async_copy(src_i.at[c, s, pl.ds(g3 * _GC, _GC)],
                             src_v.at[g3 % 2], sem_i)
            pltpu.async_copy(dst_i.at[s, pl.ds(g3 * _GC, _GC)],
                             dst_v.at[g3 % 2], sem_i)

    pltpu.make_async_copy(
        rows_a, agg_sh.at[dst_v.at[((_CH - 2) // _GC) % 2, (_CH - 2) % _GC]],
        sem_sa).wait()
    pltpu.make_async_copy(
        rows_b, agg_sh.at[dst_v.at[((_CH - 1) // _GC) % 2, (_CH - 1) % _GC]],
        sem_sb).wait()
    plsc.subcore_barrier()

    pltpu.sync_copy(agg_sh.at[pl.ds(r0, _RPT)], out.at[c, pl.ds(r0, _RPT)])

    @pl.when(s == 0)
    def _():
        pltpu.sync_copy(agg_sh.at[pl.ds(_TILES * _RPT, _RREM)],
                        out.at[c, pl.ds(_TILES * _RPT, _RREM)])


@functools.lru_cache(maxsize=None)
def _sc_agg_call():
    mesh = plsc.VectorSubcoreMesh(core_axis_name="c", subcore_axis_name="s")
    return pl.kernel(
        _sc_agg_body,
        out_type=jax.ShapeDtypeStruct((2, _N, 128), jnp.float32),
        mesh=mesh,
        scratch_types=[
            pltpu.VMEM((2, _GC, _K), jnp.int32),
            pltpu.VMEM((2, _GC, _K), jnp.int32),
            pltpu.VMEM((_K, 128), jnp.float32),
            pltpu.VMEM((_K, 128), jnp.float32),
            pltpu.VMEM_SHARED((_N, 128), jnp.float32),
            pltpu.SemaphoreType.DMA,
            pltpu.SemaphoreType.DMA,
            pltpu.SemaphoreType.DMA,
            pltpu.SemaphoreType.DMA,
            pltpu.SemaphoreType.DMA,
        ],
    )


_BR = 1000


def _mlp_blk(u, w1, b1, w2, b2):
    t = jnp.maximum(
        jnp.dot(u, w1[...], preferred_element_type=jnp.float32) + b1[...], 0.0)
    return jnp.dot(t, w2[...], preferred_element_type=jnp.float32) + b2[...]


def _l1_body(x_ref, xs_ref, af_ref, as_ref, fw1, fb1, fw2, fb2,
             sw1, sb1, sw2, sb2, flo, fhi, slo, shi):
    of = jnp.maximum(
        _mlp_blk(x_ref[...] + af_ref[...], fw1, fb1, fw2, fb2), 0.0)
    flo[...] = of[:, :128]
    fhi[...] = of[:, 128:]
    os = jnp.maximum(
        _mlp_blk(xs_ref[...] + as_ref[...], sw1, sb1, sw2, sb2), 0.0)
    slo[...] = os[:, :128]
    shi[...] = os[:, 128:]


def _layer1(x, x_s, af, a_s, pf, ps):
    row = pl.BlockSpec((_BR, 128), lambda i: (i, 0))
    wspec = lambda shp: pl.BlockSpec(shp, lambda i: (0, 0))
    return pl.pallas_call(
        _l1_body,
        grid=(_N // _BR,),
        in_specs=[row, row, row, row,
                  wspec((128, 256)), wspec((1, 256)),
                  wspec((256, 256)), wspec((1, 256)),
                  wspec((128, 256)), wspec((1, 256)),
                  wspec((256, 256)), wspec((1, 256))],
        out_specs=[row] * 4,
        out_shape=[jax.ShapeDtypeStruct((_N, 128), jnp.float32)] * 4,
    )(x, x_s, af, a_s,
      pf[0], pf[1].reshape(1, -1), pf[2], pf[3].reshape(1, -1),
      ps[0], ps[1].reshape(1, -1), ps[2], ps[3].reshape(1, -1))


def _l2_body(flo_ref, fhi_ref, slo_ref, shi_ref, aflo, afhi, aslo, ashi,
             fw1, fb1, fw2, fb2, sw1, sb1, sw2, sb2,
             oflo, ofhi, oslo, oshi):
    uf = jnp.concatenate(
        [flo_ref[...] + aflo[...], fhi_ref[...] + afhi[...]], axis=1)
    of = jnp.maximum(_mlp_blk(uf, fw1, fb1, fw2, fb2), 0.0)
    oflo[...] = of[:, :128]
    ofhi[...] = of[:, 128:]
    us = jnp.concatenate(
        [slo_ref[...] + aslo[...], shi_ref[...] + ashi[...]], axis=1)
    os = jnp.maximum(_mlp_blk(us, sw1, sb1, sw2, sb2), 0.0)
    oslo[...] = os[:, :128]
    oshi[...] = os[:, 128:]


def _layer2(flo, fhi, slo, shi, aflo, afhi, aslo, ashi, pf, ps):
    row = pl.BlockSpec((_BR, 128), lambda i: (i, 0))
    wspec = lambda shp: pl.BlockSpec(shp, lambda i: (0, 0))
    return pl.pallas_call(
        _l2_body,
        grid=(_N // _BR,),
        in_specs=[row] * 8 + [
            wspec((256, 256)), wspec((1, 256)),
            wspec((256, 256)), wspec((1, 256)),
            wspec((256, 256)), wspec((1, 256)),
            wspec((256, 256)), wspec((1, 256))],
        out_specs=[row] * 4,
        out_shape=[jax.ShapeDtypeStruct((_N, 128), jnp.float32)] * 4,
    )(flo, fhi, slo, shi, aflo, afhi, aslo, ashi,
      pf[0], pf[1].reshape(1, -1), pf[2], pf[3].reshape(1, -1),
      ps[0], ps[1].reshape(1, -1), ps[2], ps[3].reshape(1, -1))


def _nheads_body(batch_ref, f1lo, f1hi, f2lo, f2hi, s1lo, s1hi, s2lo, s2hi,
                 fw1, fb1, fw2, fb2, sw1, sb1, sw2, sb2,
                 nf_ref, ns_ref, gf_ref, gs_ref):
    i = pl.program_id(0)
    ncf = jnp.concatenate(
        [f1lo[...], f1hi[...], f2lo[...], f2hi[...]], axis=1)
    ncs = jnp.concatenate(
        [s1lo[...], s1hi[...], s2lo[...], s2hi[...]], axis=1)
    nf_ref[...] = _mlp_blk(ncf, fw1, fb1, fw2, fb2)
    ns_ref[...] = _mlp_blk(ncs, sw1, sb1, sw2, sb2)

    bm = batch_ref[0]
    gi = lax.broadcasted_iota(jnp.int32, (_G, 1), 0)
    onehot = (gi == bm).astype(jnp.float32)

    @pl.when(i == 0)
    def _():
        gf_ref[...] = jnp.zeros_like(gf_ref)
        gs_ref[...] = jnp.zeros_like(gs_ref)

    gf_ref[...] += jnp.dot(onehot, ncf, preferred_element_type=jnp.float32)
    gs_ref[...] += jnp.dot(onehot, ncs, preferred_element_type=jnp.float32)


def _nheads(batch3, hf, hs, pf, ps):
    row = pl.BlockSpec((_BR, 128), lambda i: (i, 0))
    wspec = lambda shp: pl.BlockSpec(shp, lambda i: (0, 0))
    big = pl.BlockSpec((_BR, 512), lambda i: (i, 0))
    acc = pl.BlockSpec((_G, 512), lambda i: (0, 0))
    return pl.pallas_call(
        _nheads_body,
        grid=(_N // _BR,),
        in_specs=[pl.BlockSpec((1, 1, _BR), lambda i: (i, 0, 0))]
        + [row] * 8 + [
            wspec((512, 512)), wspec((1, 512)),
            wspec((512, 512)), wspec((1, 512)),
            wspec((512, 512)), wspec((1, 512)),
            wspec((512, 512)), wspec((1, 512))],
        out_specs=[big, big, acc, acc],
        out_shape=[jax.ShapeDtypeStruct((_N, 512), jnp.float32)] * 2
        + [jax.ShapeDtypeStruct((_G, 512), jnp.float32)] * 2,
    )(batch3, *hf, *hs,
      pf[0], pf[1].reshape(1, -1), pf[2], pf[3].reshape(1, -1),
      ps[0], ps[1].reshape(1, -1), ps[2], ps[3].reshape(1, -1))


def _gheads_body(gf_ref, gs_ref, bw1, bb1, bw2, bb2,
                 fw1, fb1, fw2, fb2, sw1, sb1, sw2, sb2,
                 b_ref, ogf_ref, ogs_ref):
    gcat = jnp.concatenate([gf_ref[...], gs_ref[...]], axis=1)
    b_ref[...] = _mlp_blk(gcat, bw1, bb1, bw2, bb2)
    ogf_ref[...] = _mlp_blk(gf_ref[...], fw1, fb1, fw2, fb2)
    ogs_ref[...] = _mlp_blk(gs_ref[...], sw1, sb1, sw2, sb2)


def _gheads(gf, gs, pb, pf, ps):
    wspec = lambda shp: pl.BlockSpec(shp, lambda i: (0, 0))
    g = pl.BlockSpec((_G, 512), lambda i: (0, 0))
    return pl.pallas_call(
        _gheads_body,
        grid=(1,),
        in_specs=[g, g,
                  wspec((1024, 512)), wspec((1, 512)),
                  wspec((512, 512)), wspec((1, 512)),
                  wspec((512, 512)), wspec((1, 512)),
                  wspec((512, 512)), wspec((1, 512)),
                  wspec((512, 512)), wspec((1, 512)),
                  wspec((512, 512)), wspec((1, 512))],
        out_specs=[g, g, g],
        out_shape=[jax.ShapeDtypeStruct((_G, 512), jnp.float32)] * 3,
    )(gf, gs,
      pb[0], pb[1].reshape(1, -1), pb[2], pb[3].reshape(1, -1),
      pf[0], pf[1].reshape(1, -1), pf[2], pf[3].reshape(1, -1),
      ps[0], ps[1].reshape(1, -1), ps[2], ps[3].reshape(1, -1))


def kernel(x, x_s, params, edge_index, batch):
    src = edge_index[0].astype(jnp.int32).reshape(_TILES, _CH, _K)
    src2 = jnp.stack([src, src + _N])
    dst = edge_index[1].astype(jnp.int32).reshape(_TILES, _CH, _K)
    zeros = jnp.zeros((_N, 128), jnp.float32)
    batch3 = batch.astype(jnp.int32).reshape(_N // 1000, 1, 1000)
    agg = _sc_agg_call()

    a1 = agg(jnp.concatenate([x, x_s], axis=0), src2, dst, zeros)
    h1f_lo, h1f_hi, h1s_lo, h1s_hi = _layer1(
        x, x_s, a1[0], a1[1], params["ef"][0], params["es"][0])

    a2f = agg(jnp.concatenate([h1f_lo, h1f_hi], axis=0), src2, dst, zeros)
    a2s = agg(jnp.concatenate([h1s_lo, h1s_hi], axis=0), src2, dst, zeros)
    h2f_lo, h2f_hi, h2s_lo, h2s_hi = _layer2(
        h1f_lo, h1f_hi, h1s_lo, h1s_hi, a2f[0], a2f[1], a2s[0], a2s[1],
        params["ef"][1], params["es"][1])

    n_f, n_s, g_f_raw, g_s_raw = _nheads(
        batch3, (h1f_lo, h1f_hi, h2f_lo, h2f_hi),
        (h1s_lo, h1s_hi, h2s_lo, h2s_hi), params["pfn"], params["psn"])
    b, g_f, g_s = _gheads(g_f_raw, g_s_raw,
                          params["pb"], params["pfg"], params["psg"])
    return (b, g_f, g_s, n_f, n_s)


# speedup vs baseline: 6.5505x; 1.0027x over previous
import functools

import jax
import jax.numpy as jnp
from jax import lax
from jax.experimental import pallas as pl
from jax.experimental.pallas import tpu as pltpu
from jax.experimental.pallas import tpu_sc as plsc

_N = 10000
_E = 320000
_G = 128
_TILES = 16
_K = 125
_CH = _E // _TILES // _K
_GC = 32
_NG = _CH // _GC
_RPT = 624
_RREM = _N - _TILES * _RPT


def _sc_agg_body(tab, src_i, dst_i, zeros, out, src_v, dst_v, rows_a, rows_b,
                 agg_sh, sem_a, sem_b, sem_i, sem_sa, sem_sb):
    c = lax.axis_index("c")
    s = lax.axis_index("s")
    r0 = s * _RPT
    pltpu.sync_copy(zeros.at[pl.ds(r0, _RPT)], agg_sh.at[pl.ds(r0, _RPT)])

    @pl.when(s == 0)
    def _():
        pltpu.sync_copy(zeros.at[pl.ds(_TILES * _RPT, _RREM)],
                        agg_sh.at[pl.ds(_TILES * _RPT, _RREM)])

    pltpu.sync_copy(src_i.at[c, s, pl.ds(0, _GC)], src_v.at[0])
    pltpu.sync_copy(dst_i.at[s, pl.ds(0, _GC)], dst_v.at[0])
    plsc.subcore_barrier()
    pltpu.async_copy(src_i.at[c, s, pl.ds(_GC, _GC)], src_v.at[1], sem_i)
    pltpu.async_copy(dst_i.at[s, pl.ds(_GC, _GC)], dst_v.at[1], sem_i)
    pltpu.async_copy(tab.at[src_v.at[0, 0]], rows_a, sem_a)
    pltpu.async_copy(tab.at[src_v.at[0, 1]], rows_b, sem_b)

    @pl.loop(0, _CH // 2)
    def _pair(p):
        ch0 = 2 * p
        ch1 = ch0 + 1
        ch2 = ch0 + 2
        ch3 = ch0 + 3
        s0 = (ch0 // _GC) % 2
        s2 = (ch2 // _GC) % 2

        pltpu.make_async_copy(tab.at[src_v.at[s0, ch0 % _GC]],
                              rows_a, sem_a).wait()
        pltpu.async_copy(rows_a, agg_sh.at[dst_v.at[s0, ch0 % _GC]],
                         sem_sa, add=True)

        @pl.when(jnp.logical_and(ch2 % _GC == 0, ch2 < _CH))
        def _():
            pltpu.make_async_copy(src_i.at[c, s, pl.ds(0, _GC)],
                                  src_v.at[0], sem_i).wait()
            pltpu.make_async_copy(dst_i.at[s, pl.ds(0, _GC)],
                                  dst_v.at[0], sem_i).wait()

        @pl.when(ch2 < _CH)
        def _():
            pltpu.make_async_copy(rows_a,
                                  agg_sh.at[dst_v.at[s0, ch0 % _GC]],
                                  sem_sa).wait()
            pltpu.async_copy(tab.at[src_v.at[s2, ch2 % _GC]], rows_a, sem_a)

        pltpu.make_async_copy(tab.at[src_v.at[s0, ch1 % _GC]],
                              rows_b, sem_b).wait()
        pltpu.async_copy(rows_b, agg_sh.at[dst_v.at[s0, ch1 % _GC]],
                         sem_sb, add=True)

        @pl.when(ch3 < _CH)
        def _():
            pltpu.make_async_copy(rows_b,
                                  agg_sh.at[dst_v.at[s0, ch1 % _GC]],
                                  sem_sb).wait()
            pltpu.async_copy(tab.at[src_v.at[s2, ch3 % _GC]], rows_b, sem_b)

        @pl.when(jnp.logical_and(ch2 % _GC == 0, ch2 + _GC < _CH))
        def _():
            g3 = ch2 // _GC + 1
            pltpu.async_copy(src_i.at[c, s, pl.ds(g3 * _GC, _GC)],
                             src_v.at[g3 % 2], sem_i)
            pltpu.async_copy(dst_i.at[s, pl.ds(g3 * _GC, _GC)],
                             dst_v.at[g3 % 2], sem_i)

    pltpu.make_async_copy(
        rows_a, agg_sh.at[dst_v.at[((_CH - 2) // _GC) % 2, (_CH - 2) % _GC]],
        sem_sa).wait()
    pltpu.make_async_copy(
        rows_b, agg_sh.at[dst_v.at[((_CH - 1) // _GC) % 2, (_CH - 1) % _GC]],
        sem_sb).wait()
    plsc.subcore_barrier()

    pltpu.sync_copy(agg_sh.at[pl.ds(r0, _RPT)], out.at[c, pl.ds(r0, _RPT)])

    @pl.when(s == 0)
    def _():
        pltpu.sync_copy(agg_sh.at[pl.ds(_TILES * _RPT, _RREM)],
                        out.at[c, pl.ds(_TILES * _RPT, _RREM)])


@functools.lru_cache(maxsize=None)
def _sc_agg_call():
    mesh = plsc.VectorSubcoreMesh(core_axis_name="c", subcore_axis_name="s")
    return pl.kernel(
        _sc_agg_body,
        out_type=jax.ShapeDtypeStruct((2, _N, 128), jnp.float32),
        mesh=mesh,
        scratch_types=[
            pltpu.VMEM((2, _GC, _K), jnp.int32),
            pltpu.VMEM((2, _GC, _K), jnp.int32),
            pltpu.VMEM((_K, 128), jnp.float32),
            pltpu.VMEM((_K, 128), jnp.float32),
            pltpu.VMEM_SHARED((_N, 128), jnp.float32),
            pltpu.SemaphoreType.DMA,
            pltpu.SemaphoreType.DMA,
            pltpu.SemaphoreType.DMA,
            pltpu.SemaphoreType.DMA,
            pltpu.SemaphoreType.DMA,
        ],
    )


_BR = 1000


def _mlp_blk(u, w1, b1, w2, b2):
    t = jnp.maximum(
        jnp.dot(u, w1[...], preferred_element_type=jnp.float32) + b1[...], 0.0)
    return jnp.dot(t, w2[...], preferred_element_type=jnp.float32) + b2[...]


def _l1_body(x_ref, a_ref, w1, b1, w2, b2, lo, hi):
    o = jnp.maximum(
        _mlp_blk(x_ref[...] + a_ref[...], w1, b1, w2, b2), 0.0)
    lo[...] = o[:, :128]
    hi[...] = o[:, 128:]


def _layer1_one(x, a, p):
    row = pl.BlockSpec((_BR, 128), lambda i: (i, 0))
    wspec = lambda shp: pl.BlockSpec(shp, lambda i: (0, 0))
    return pl.pallas_call(
        _l1_body,
        grid=(_N // _BR,),
        in_specs=[row, row,
                  wspec((128, 256)), wspec((1, 256)),
                  wspec((256, 256)), wspec((1, 256))],
        out_specs=[row] * 2,
        out_shape=[jax.ShapeDtypeStruct((_N, 128), jnp.float32)] * 2,
    )(x, a, p[0], p[1].reshape(1, -1), p[2], p[3].reshape(1, -1))


def _l2_body(hlo_ref, hhi_ref, alo, ahi, w1, b1, w2, b2, olo, ohi):
    u = jnp.concatenate(
        [hlo_ref[...] + alo[...], hhi_ref[...] + ahi[...]], axis=1)
    o = jnp.maximum(_mlp_blk(u, w1, b1, w2, b2), 0.0)
    olo[...] = o[:, :128]
    ohi[...] = o[:, 128:]


def _layer2_one(hlo, hhi, alo, ahi, p):
    row = pl.BlockSpec((_BR, 128), lambda i: (i, 0))
    wspec = lambda shp: pl.BlockSpec(shp, lambda i: (0, 0))
    return pl.pallas_call(
        _l2_body,
        grid=(_N // _BR,),
        in_specs=[row] * 4 + [
            wspec((256, 256)), wspec((1, 256)),
            wspec((256, 256)), wspec((1, 256))],
        out_specs=[row] * 2,
        out_shape=[jax.ShapeDtypeStruct((_N, 128), jnp.float32)] * 2,
    )(hlo, hhi, alo, ahi,
      p[0], p[1].reshape(1, -1), p[2], p[3].reshape(1, -1))


def _nheads_body(batch_ref, h1lo, h1hi, h2lo, h2hi, w1, b1, w2, b2,
                 n_ref, g_ref):
    i = pl.program_id(0)
    nc = jnp.concatenate(
        [h1lo[...], h1hi[...], h2lo[...], h2hi[...]], axis=1)
    n_ref[...] = _mlp_blk(nc, w1, b1, w2, b2)

    bm = batch_ref[0]
    gi = lax.broadcasted_iota(jnp.int32, (_G, 1), 0)
    onehot = (gi == bm).astype(jnp.float32)

    @pl.when(i == 0)
    def _():
        g_ref[...] = jnp.zeros_like(g_ref)

    g_ref[...] += jnp.dot(onehot, nc, preferred_element_type=jnp.float32)


def _nheads_one(batch3, h1lo, h1hi, h2lo, h2hi, p):
    row = pl.BlockSpec((_BR, 128), lambda i: (i, 0))
    wspec = lambda shp: pl.BlockSpec(shp, lambda i: (0, 0))
    big = pl.BlockSpec((_BR, 512), lambda i: (i, 0))
    acc = pl.BlockSpec((_G, 512), lambda i: (0, 0))
    return pl.pallas_call(
        _nheads_body,
        grid=(_N // _BR,),
        in_specs=[pl.BlockSpec((1, 1, _BR), lambda i: (i, 0, 0))]
        + [row] * 4 + [
            wspec((512, 512)), wspec((1, 512)),
            wspec((512, 512)), wspec((1, 512))],
        out_specs=[big, acc],
        out_shape=[jax.ShapeDtypeStruct((_N, 512), jnp.float32),
                   jax.ShapeDtypeStruct((_G, 512), jnp.float32)],
    )(batch3, h1lo, h1hi, h2lo, h2hi,
      p[0], p[1].reshape(1, -1), p[2], p[3].reshape(1, -1))


def _gheads_body(gf_ref, gs_ref, bw1, bb1, bw2, bb2,
                 fw1, fb1, fw2, fb2, sw1, sb1, sw2, sb2,
                 b_ref, ogf_ref, ogs_ref):
    gcat = jnp.concatenate([gf_ref[...], gs_ref[...]], axis=1)
    b_ref[...] = _mlp_blk(gcat, bw1, bb1, bw2, bb2)
    ogf_ref[...] = _mlp_blk(gf_ref[...], fw1, fb1, fw2, fb2)
    ogs_ref[...] = _mlp_blk(gs_ref[...], sw1, sb1, sw2, sb2)


def _gheads(gf, gs, pb, pf, ps):
    wspec = lambda shp: pl.BlockSpec(shp, lambda i: (0, 0))
    g = pl.BlockSpec((_G, 512), lambda i: (0, 0))
    return pl.pallas_call(
        _gheads_body,
        grid=(1,),
        in_specs=[g, g,
                  wspec((1024, 512)), wspec((1, 512)),
                  wspec((512, 512)), wspec((1, 512)),
                  wspec((512, 512)), wspec((1, 512)),
                  wspec((512, 512)), wspec((1, 512)),
                  wspec((512, 512)), wspec((1, 512)),
                  wspec((512, 512)), wspec((1, 512))],
        out_specs=[g, g, g],
        out_shape=[jax.ShapeDtypeStruct((_G, 512), jnp.float32)] * 3,
    )(gf, gs,
      pb[0], pb[1].reshape(1, -1), pb[2], pb[3].reshape(1, -1),
      pf[0], pf[1].reshape(1, -1), pf[2], pf[3].reshape(1, -1),
      ps[0], ps[1].reshape(1, -1), ps[2], ps[3].reshape(1, -1))


def kernel(x, x_s, params, edge_index, batch):
    src = edge_index[0].astype(jnp.int32).reshape(_TILES, _CH, _K)
    src2 = jnp.stack([src, src + _N])
    dst = edge_index[1].astype(jnp.int32).reshape(_TILES, _CH, _K)
    zeros = jnp.zeros((_N, 128), jnp.float32)
    batch3 = batch.astype(jnp.int32).reshape(_N // 1000, 1, 1000)
    agg = _sc_agg_call()

    a1 = agg(jnp.concatenate([x, x_s], axis=0), src2, dst, zeros)
    h1f_lo, h1f_hi = _layer1_one(x, a1[0], params["ef"][0])
    a2f = agg(jnp.concatenate([h1f_lo, h1f_hi], axis=0), src2, dst, zeros)
    h1s_lo, h1s_hi = _layer1_one(x_s, a1[1], params["es"][0])
    a2s = agg(jnp.concatenate([h1s_lo, h1s_hi], axis=0), src2, dst, zeros)
    h2f_lo, h2f_hi = _layer2_one(h1f_lo, h1f_hi, a2f[0], a2f[1],
                                 params["ef"][1])
    n_f, g_f_raw = _nheads_one(batch3, h1f_lo, h1f_hi, h2f_lo, h2f_hi,
                               params["pfn"])
    h2s_lo, h2s_hi = _layer2_one(h1s_lo, h1s_hi, a2s[0], a2s[1],
                                 params["es"][1])
    n_s, g_s_raw = _nheads_one(batch3, h1s_lo, h1s_hi, h2s_lo, h2s_hi,
                               params["psn"])
    b, g_f, g_s = _gheads(g_f_raw, g_s_raw,
                          params["pb"], params["pfg"], params["psg"])
    return (b, g_f, g_s, n_f, n_s)


# speedup vs baseline: 6.8442x; 1.0448x over previous
import functools

import jax
import jax.numpy as jnp
from jax import lax
from jax.experimental import pallas as pl
from jax.experimental.pallas import tpu as pltpu
from jax.experimental.pallas import tpu_sc as plsc

_N = 10000
_E = 320000
_G = 128
_TILES = 16
_K = 125
_CH = _E // _TILES // _K
_GC = 32
_NG = _CH // _GC
_RPT = 624
_RREM = _N - _TILES * _RPT


def _sc_agg_body(tab, src_i, dst_i, zeros, out, src_v, dst_v, rows_a, rows_b,
                 agg_sh, sem_a, sem_b, sem_i, sem_sa, sem_sb):
    c = lax.axis_index("c")
    s = lax.axis_index("s")
    r0 = s * _RPT
    pltpu.sync_copy(zeros.at[pl.ds(r0, _RPT)], agg_sh.at[pl.ds(r0, _RPT)])

    @pl.when(s == 0)
    def _():
        pltpu.sync_copy(zeros.at[pl.ds(_TILES * _RPT, _RREM)],
                        agg_sh.at[pl.ds(_TILES * _RPT, _RREM)])

    pltpu.sync_copy(src_i.at[c, s, pl.ds(0, _GC)], src_v.at[0])
    pltpu.sync_copy(dst_i.at[s, pl.ds(0, _GC)], dst_v.at[0])
    plsc.subcore_barrier()
    pltpu.async_copy(src_i.at[c, s, pl.ds(_GC, _GC)], src_v.at[1], sem_i)
    pltpu.async_copy(dst_i.at[s, pl.ds(_GC, _GC)], dst_v.at[1], sem_i)
    pltpu.async_copy(tab.at[src_v.at[0, 0]], rows_a, sem_a)
    pltpu.async_copy(tab.at[src_v.at[0, 1]], rows_b, sem_b)

    @pl.loop(0, _CH // 2)
    def _pair(p):
        ch0 = 2 * p
        ch1 = ch0 + 1
        ch2 = ch0 + 2
        ch3 = ch0 + 3
        s0 = (ch0 // _GC) % 2
        s2 = (ch2 // _GC) % 2

        pltpu.make_async_copy(tab.at[src_v.at[s0, ch0 % _GC]],
                              rows_a, sem_a).wait()
        pltpu.async_copy(rows_a, agg_sh.at[dst_v.at[s0, ch0 % _GC]],
                         sem_sa, add=True)

        @pl.when(jnp.logical_and(ch2 % _GC == 0, ch2 < _CH))
        def _():
            pltpu.make_async_copy(src_i.at[c, s, pl.ds(0, _GC)],
                                  src_v.at[0], sem_i).wait()
            pltpu.make_async_copy(dst_i.at[s, pl.ds(0, _GC)],
                                  dst_v.at[0], sem_i).wait()

        @pl.when(ch2 < _CH)
        def _():
            pltpu.make_async_copy(rows_a,
                                  agg_sh.at[dst_v.at[s0, ch0 % _GC]],
                                  sem_sa).wait()
            pltpu.async_copy(tab.at[src_v.at[s2, ch2 % _GC]], rows_a, sem_a)

        pltpu.make_async_copy(tab.at[src_v.at[s0, ch1 % _GC]],
                              rows_b, sem_b).wait()
        pltpu.async_copy(rows_b, agg_sh.at[dst_v.at[s0, ch1 % _GC]],
                         sem_sb, add=True)

        @pl.when(ch3 < _CH)
        def _():
            pltpu.make_async_copy(rows_b,
                                  agg_sh.at[dst_v.at[s0, ch1 % _GC]],
                                  sem_sb).wait()
            pltpu.async_copy(tab.at[src_v.at[s2, ch3 % _GC]], rows_b, sem_b)

        @pl.when(jnp.logical_and(ch2 % _GC == 0, ch2 + _GC < _CH))
        def _():
            g3 = ch2 // _GC + 1
            pltpu.async_copy(src_i.at[c, s, pl.ds(g3 * _GC, _GC)],
                             src_v.at[g3 % 2], sem_i)
            pltpu.async_copy(dst_i.at[s, pl.ds(g3 * _GC, _GC)],
                             dst_v.at[g3 % 2], sem_i)

    pltpu.make_async_copy(
        rows_a, agg_sh.at[dst_v.at[((_CH - 2) // _GC) % 2, (_CH - 2) % _GC]],
        sem_sa).wait()
    pltpu.make_async_copy(
        rows_b, agg_sh.at[dst_v.at[((_CH - 1) // _GC) % 2, (_CH - 1) % _GC]],
        sem_sb).wait()
    plsc.subcore_barrier()

    pltpu.sync_copy(agg_sh.at[pl.ds(r0, _RPT)], out.at[c, pl.ds(r0, _RPT)])

    @pl.when(s == 0)
    def _():
        pltpu.sync_copy(agg_sh.at[pl.ds(_TILES * _RPT, _RREM)],
                        out.at[c, pl.ds(_TILES * _RPT, _RREM)])


@functools.lru_cache(maxsize=None)
def _sc_agg_call():
    mesh = plsc.VectorSubcoreMesh(core_axis_name="c", subcore_axis_name="s")
    return pl.kernel(
        _sc_agg_body,
        out_type=jax.ShapeDtypeStruct((2, _N, 128), jnp.float32),
        mesh=mesh,
        scratch_types=[
            pltpu.VMEM((2, _GC, _K), jnp.int32),
            pltpu.VMEM((2, _GC, _K), jnp.int32),
            pltpu.VMEM((_K, 128), jnp.float32),
            pltpu.VMEM((_K, 128), jnp.float32),
            pltpu.VMEM_SHARED((_N, 128), jnp.float32),
            pltpu.SemaphoreType.DMA,
            pltpu.SemaphoreType.DMA,
            pltpu.SemaphoreType.DMA,
            pltpu.SemaphoreType.DMA,
            pltpu.SemaphoreType.DMA,
        ],
    )


_BR = 1000


def _mlp_blk(u, w1, b1, w2, b2):
    t = jnp.maximum(
        jnp.dot(u, w1[...], preferred_element_type=jnp.float32) + b1[...], 0.0)
    return jnp.dot(t, w2[...], preferred_element_type=jnp.float32) + b2[...]


def _l1_body(x_ref, a_ref, w1, b1, w2, b2, lo, hi):
    o = jnp.maximum(
        _mlp_blk(x_ref[...] + a_ref[...], w1, b1, w2, b2), 0.0)
    lo[...] = o[:, :128]
    hi[...] = o[:, 128:]


def _layer1_one(x, a, p):
    row = pl.BlockSpec((_BR, 128), lambda i: (i, 0))
    wspec = lambda shp: pl.BlockSpec(shp, lambda i: (0, 0))
    return pl.pallas_call(
        _l1_body,
        grid=(_N // _BR,),
        in_specs=[row, row,
                  wspec((128, 256)), wspec((1, 256)),
                  wspec((256, 256)), wspec((1, 256))],
        out_specs=[row] * 2,
        out_shape=[jax.ShapeDtypeStruct((_N, 128), jnp.float32)] * 2,
    )(x, a, p[0], p[1].reshape(1, -1), p[2], p[3].reshape(1, -1))


def _tail_body(batch_ref, h1lo, h1hi, alo, ahi,
               lw1, lb1, lw2, lb2, w1, b1, w2, b2, n_ref, g_ref):
    i = pl.program_id(0)
    u = jnp.concatenate(
        [h1lo[...] + alo[...], h1hi[...] + ahi[...]], axis=1)
    h2 = jnp.maximum(_mlp_blk(u, lw1, lb1, lw2, lb2), 0.0)
    nc = jnp.concatenate([h1lo[...], h1hi[...], h2], axis=1)
    n_ref[...] = _mlp_blk(nc, w1, b1, w2, b2)

    bm = batch_ref[0]
    gi = lax.broadcasted_iota(jnp.int32, (_G, 1), 0)
    onehot = (gi == bm).astype(jnp.float32)

    @pl.when(i == 0)
    def _():
        g_ref[...] = jnp.zeros_like(g_ref)

    g_ref[...] += jnp.dot(onehot, nc, preferred_element_type=jnp.float32)


def _tail_one(batch3, h1lo, h1hi, alo, ahi, pl2, p):
    row = pl.BlockSpec((_BR, 128), lambda i: (i, 0))
    wspec = lambda shp: pl.BlockSpec(shp, lambda i: (0, 0))
    big = pl.BlockSpec((_BR, 512), lambda i: (i, 0))
    acc = pl.BlockSpec((_G, 512), lambda i: (0, 0))
    return pl.pallas_call(
        _tail_body,
        grid=(_N // _BR,),
        in_specs=[pl.BlockSpec((1, 1, _BR), lambda i: (i, 0, 0))]
        + [row] * 4 + [
            wspec((256, 256)), wspec((1, 256)),
            wspec((256, 256)), wspec((1, 256)),
            wspec((512, 512)), wspec((1, 512)),
            wspec((512, 512)), wspec((1, 512))],
        out_specs=[big, acc],
        out_shape=[jax.ShapeDtypeStruct((_N, 512), jnp.float32),
                   jax.ShapeDtypeStruct((_G, 512), jnp.float32)],
    )(batch3, h1lo, h1hi, alo, ahi,
      pl2[0], pl2[1].reshape(1, -1), pl2[2], pl2[3].reshape(1, -1),
      p[0], p[1].reshape(1, -1), p[2], p[3].reshape(1, -1))


def _gheads_body(gf_ref, gs_ref, bw1, bb1, bw2, bb2,
                 fw1, fb1, fw2, fb2, sw1, sb1, sw2, sb2,
                 b_ref, ogf_ref, ogs_ref):
    gcat = jnp.concatenate([gf_ref[...], gs_ref[...]], axis=1)
    b_ref[...] = _mlp_blk(gcat, bw1, bb1, bw2, bb2)
    ogf_ref[...] = _mlp_blk(gf_ref[...], fw1, fb1, fw2, fb2)
    ogs_ref[...] = _mlp_blk(gs_ref[...], sw1, sb1, sw2, sb2)


def _gheads(gf, gs, pb, pf, ps):
    wspec = lambda shp: pl.BlockSpec(shp, lambda i: (0, 0))
    g = pl.BlockSpec((_G, 512), lambda i: (0, 0))
    return pl.pallas_call(
        _gheads_body,
        grid=(1,),
        in_specs=[g, g,
                  wspec((1024, 512)), wspec((1, 512)),
                  wspec((512, 512)), wspec((1, 512)),
                  wspec((512, 512)), wspec((1, 512)),
                  wspec((512, 512)), wspec((1, 512)),
                  wspec((512, 512)), wspec((1, 512)),
                  wspec((512, 512)), wspec((1, 512))],
        out_specs=[g, g, g],
        out_shape=[jax.ShapeDtypeStruct((_G, 512), jnp.float32)] * 3,
    )(gf, gs,
      pb[0], pb[1].reshape(1, -1), pb[2], pb[3].reshape(1, -1),
      pf[0], pf[1].reshape(1, -1), pf[2], pf[3].reshape(1, -1),
      ps[0], ps[1].reshape(1, -1), ps[2], ps[3].reshape(1, -1))


def kernel(x, x_s, params, edge_index, batch):
    src = edge_index[0].astype(jnp.int32).reshape(_TILES, _CH, _K)
    src2 = jnp.stack([src, src + _N])
    dst = edge_index[1].astype(jnp.int32).reshape(_TILES, _CH, _K)
    zeros = jnp.zeros((_N, 128), jnp.float32)
    batch3 = batch.astype(jnp.int32).reshape(_N // 1000, 1, 1000)
    agg = _sc_agg_call()

    a1 = agg(jnp.concatenate([x, x_s], axis=0), src2, dst, zeros)
    h1f_lo, h1f_hi = _layer1_one(x, a1[0], params["ef"][0])
    a2f = agg(jnp.concatenate([h1f_lo, h1f_hi], axis=0), src2, dst, zeros)
    h1s_lo, h1s_hi = _layer1_one(x_s, a1[1], params["es"][0])
    a2s = agg(jnp.concatenate([h1s_lo, h1s_hi], axis=0), src2, dst, zeros)
    n_f, g_f_raw = _tail_one(batch3, h1f_lo, h1f_hi, a2f[0], a2f[1],
                             params["ef"][1], params["pfn"])
    n_s, g_s_raw = _tail_one(batch3, h1s_lo, h1s_hi, a2s[0], a2s[1],
                             params["es"][1], params["psn"])
    b, g_f, g_s = _gheads(g_f_raw, g_s_raw,
                          params["pb"], params["pfg"], params["psg"])
    return (b, g_f, g_s, n_f, n_s)


# speedup vs baseline: 6.9918x; 1.0216x over previous
import functools

import jax
import jax.numpy as jnp
from jax import lax
from jax.experimental import pallas as pl
from jax.experimental.pallas import tpu as pltpu
from jax.experimental.pallas import tpu_sc as plsc

_N = 10000
_E = 320000
_G = 128
_TILES = 16
_K = 125
_CH = _E // _TILES // _K
_GC = 32
_NG = _CH // _GC
_RPT = 624
_RREM = _N - _TILES * _RPT


def _sc_agg_body(tab, ei, zeros, out, src_v, dst_v, rows_a, rows_b,
                 agg_sh, sem_a, sem_b, sem_i, sem_sa, sem_sb):
    c = lax.axis_index("c")
    s = lax.axis_index("s")
    r0 = s * _RPT
    tab_c = tab.at[pl.ds(c * _N, _N)]
    pltpu.sync_copy(zeros.at[pl.ds(r0, _RPT)], agg_sh.at[pl.ds(r0, _RPT)])

    @pl.when(s == 0)
    def _():
        pltpu.sync_copy(zeros.at[pl.ds(_TILES * _RPT, _RREM)],
                        agg_sh.at[pl.ds(_TILES * _RPT, _RREM)])

    pltpu.sync_copy(ei.at[0, s, pl.ds(0, _GC)], src_v.at[0])
    pltpu.sync_copy(ei.at[1, s, pl.ds(0, _GC)], dst_v.at[0])
    plsc.subcore_barrier()
    pltpu.async_copy(ei.at[0, s, pl.ds(_GC, _GC)], src_v.at[1], sem_i)
    pltpu.async_copy(ei.at[1, s, pl.ds(_GC, _GC)], dst_v.at[1], sem_i)
    pltpu.async_copy(tab_c.at[src_v.at[0, 0]], rows_a, sem_a)
    pltpu.async_copy(tab_c.at[src_v.at[0, 1]], rows_b, sem_b)

    @pl.loop(0, _CH // 2)
    def _pair(p):
        ch0 = 2 * p
        ch1 = ch0 + 1
        ch2 = ch0 + 2
        ch3 = ch0 + 3
        s0 = (ch0 // _GC) % 2
        s2 = (ch2 // _GC) % 2

        pltpu.make_async_copy(tab_c.at[src_v.at[s0, ch0 % _GC]],
                              rows_a, sem_a).wait()
        pltpu.async_copy(rows_a, agg_sh.at[dst_v.at[s0, ch0 % _GC]],
                         sem_sa, add=True)

        @pl.when(jnp.logical_and(ch2 % _GC == 0, ch2 < _CH))
        def _():
            pltpu.make_async_copy(ei.at[0, s, pl.ds(0, _GC)],
                                  src_v.at[0], sem_i).wait()
            pltpu.make_async_copy(ei.at[1, s, pl.ds(0, _GC)],
                                  dst_v.at[0], sem_i).wait()

        @pl.when(ch2 < _CH)
        def _():
            pltpu.make_async_copy(rows_a,
                                  agg_sh.at[dst_v.at[s0, ch0 % _GC]],
                                  sem_sa).wait()
            pltpu.async_copy(tab_c.at[src_v.at[s2, ch2 % _GC]], rows_a, sem_a)

        pltpu.make_async_copy(tab_c.at[src_v.at[s0, ch1 % _GC]],
                              rows_b, sem_b).wait()
        pltpu.async_copy(rows_b, agg_sh.at[dst_v.at[s0, ch1 % _GC]],
                         sem_sb, add=True)

        @pl.when(ch3 < _CH)
        def _():
            pltpu.make_async_copy(rows_b,
                                  agg_sh.at[dst_v.at[s0, ch1 % _GC]],
                                  sem_sb).wait()
            pltpu.async_copy(tab_c.at[src_v.at[s2, ch3 % _GC]], rows_b, sem_b)

        @pl.when(jnp.logical_and(ch2 % _GC == 0, ch2 + _GC < _CH))
        def _():
            g3 = ch2 // _GC + 1
            pltpu.async_copy(ei.at[0, s, pl.ds(g3 * _GC, _GC)],
                             src_v.at[g3 % 2], sem_i)
            pltpu.async_copy(ei.at[1, s, pl.ds(g3 * _GC, _GC)],
                             dst_v.at[g3 % 2], sem_i)

    pltpu.make_async_copy(
        rows_a, agg_sh.at[dst_v.at[((_CH - 2) // _GC) % 2, (_CH - 2) % _GC]],
        sem_sa).wait()
    pltpu.make_async_copy(
        rows_b, agg_sh.at[dst_v.at[((_CH - 1) // _GC) % 2, (_CH - 1) % _GC]],
        sem_sb).wait()
    plsc.subcore_barrier()

    pltpu.sync_copy(agg_sh.at[pl.ds(r0, _RPT)], out.at[c, pl.ds(r0, _RPT)])

    @pl.when(s == 0)
    def _():
        pltpu.sync_copy(agg_sh.at[pl.ds(_TILES * _RPT, _RREM)],
                        out.at[c, pl.ds(_TILES * _RPT, _RREM)])


@functools.lru_cache(maxsize=None)
def _sc_agg_call():
    mesh = plsc.VectorSubcoreMesh(core_axis_name="c", subcore_axis_name="s")
    return pl.kernel(
        _sc_agg_body,
        out_type=jax.ShapeDtypeStruct((2, _N, 128), jnp.float32),
        mesh=mesh,
        scratch_types=[
            pltpu.VMEM((2, _GC, _K), jnp.int32),
            pltpu.VMEM((2, _GC, _K), jnp.int32),
            pltpu.VMEM((_K, 128), jnp.float32),
            pltpu.VMEM((_K, 128), jnp.float32),
            pltpu.VMEM_SHARED((_N, 128), jnp.float32),
            pltpu.SemaphoreType.DMA,
            pltpu.SemaphoreType.DMA,
            pltpu.SemaphoreType.DMA,
            pltpu.SemaphoreType.DMA,
            pltpu.SemaphoreType.DMA,
        ],
    )


_BR = 1000


def _mlp_blk(u, w1, b1, w2, b2):
    t = jnp.maximum(
        jnp.dot(u, w1[...], preferred_element_type=jnp.float32) + b1[...], 0.0)
    return jnp.dot(t, w2[...], preferred_element_type=jnp.float32) + b2[...]


def _l1_body(x_ref, a_ref, w1, b1, w2, b2, lo, hi):
    o = jnp.maximum(
        _mlp_blk(x_ref[...] + a_ref[...], w1, b1, w2, b2), 0.0)
    lo[...] = o[:, :128]
    hi[...] = o[:, 128:]


def _layer1_one(x, a, p):
    row = pl.BlockSpec((_BR, 128), lambda i: (i, 0))
    wspec = lambda shp: pl.BlockSpec(shp, lambda i: (0, 0))
    return pl.pallas_call(
        _l1_body,
        grid=(_N // _BR,),
        in_specs=[row, row,
                  wspec((128, 256)), wspec((1, 256)),
                  wspec((256, 256)), wspec((1, 256))],
        out_specs=[row] * 2,
        out_shape=[jax.ShapeDtypeStruct((_N, 128), jnp.float32)] * 2,
    )(x, a, p[0], p[1].reshape(1, -1), p[2], p[3].reshape(1, -1))


def _tail_body(batch_ref, h1lo, h1hi, alo, ahi,
               lw1, lb1, lw2, lb2, w1, b1, w2, b2, n_ref, g_ref):
    i = pl.program_id(0)
    u = jnp.concatenate(
        [h1lo[...] + alo[...], h1hi[...] + ahi[...]], axis=1)
    h2 = jnp.maximum(_mlp_blk(u, lw1, lb1, lw2, lb2), 0.0)
    nc = jnp.concatenate([h1lo[...], h1hi[...], h2], axis=1)
    n_ref[...] = _mlp_blk(nc, w1, b1, w2, b2)

    bm = batch_ref[0]
    gi = lax.broadcasted_iota(jnp.int32, (_G, 1), 0)
    onehot = (gi == bm).astype(jnp.float32)

    @pl.when(i == 0)
    def _():
        g_ref[...] = jnp.zeros_like(g_ref)

    g_ref[...] += jnp.dot(onehot, nc, preferred_element_type=jnp.float32)


def _tail_one(batch3, h1lo, h1hi, alo, ahi, pl2, p):
    row = pl.BlockSpec((_BR, 128), lambda i: (i, 0))
    wspec = lambda shp: pl.BlockSpec(shp, lambda i: (0, 0))
    big = pl.BlockSpec((_BR, 512), lambda i: (i, 0))
    acc = pl.BlockSpec((_G, 512), lambda i: (0, 0))
    return pl.pallas_call(
        _tail_body,
        grid=(_N // _BR,),
        in_specs=[pl.BlockSpec((1, 1, _BR), lambda i: (i, 0, 0))]
        + [row] * 4 + [
            wspec((256, 256)), wspec((1, 256)),
            wspec((256, 256)), wspec((1, 256)),
            wspec((512, 512)), wspec((1, 512)),
            wspec((512, 512)), wspec((1, 512))],
        out_specs=[big, acc],
        out_shape=[jax.ShapeDtypeStruct((_N, 512), jnp.float32),
                   jax.ShapeDtypeStruct((_G, 512), jnp.float32)],
    )(batch3, h1lo, h1hi, alo, ahi,
      pl2[0], pl2[1].reshape(1, -1), pl2[2], pl2[3].reshape(1, -1),
      p[0], p[1].reshape(1, -1), p[2], p[3].reshape(1, -1))


def _gheads_body(gf_ref, gs_ref, bw1, bb1, bw2, bb2,
                 fw1, fb1, fw2, fb2, sw1, sb1, sw2, sb2,
                 b_ref, ogf_ref, ogs_ref):
    gcat = jnp.concatenate([gf_ref[...], gs_ref[...]], axis=1)
    b_ref[...] = _mlp_blk(gcat, bw1, bb1, bw2, bb2)
    ogf_ref[...] = _mlp_blk(gf_ref[...], fw1, fb1, fw2, fb2)
    ogs_ref[...] = _mlp_blk(gs_ref[...], sw1, sb1, sw2, sb2)


def _gheads(gf, gs, pb, pf, ps):
    wspec = lambda shp: pl.BlockSpec(shp, lambda i: (0, 0))
    g = pl.BlockSpec((_G, 512), lambda i: (0, 0))
    return pl.pallas_call(
        _gheads_body,
        grid=(1,),
        in_specs=[g, g,
                  wspec((1024, 512)), wspec((1, 512)),
                  wspec((512, 512)), wspec((1, 512)),
                  wspec((512, 512)), wspec((1, 512)),
                  wspec((512, 512)), wspec((1, 512)),
                  wspec((512, 512)), wspec((1, 512)),
                  wspec((512, 512)), wspec((1, 512))],
        out_specs=[g, g, g],
        out_shape=[jax.ShapeDtypeStruct((_G, 512), jnp.float32)] * 3,
    )(gf, gs,
      pb[0], pb[1].reshape(1, -1), pb[2], pb[3].reshape(1, -1),
      pf[0], pf[1].reshape(1, -1), pf[2], pf[3].reshape(1, -1),
      ps[0], ps[1].reshape(1, -1), ps[2], ps[3].reshape(1, -1))


def kernel(x, x_s, params, edge_index, batch):
    ei = edge_index.astype(jnp.int32).reshape(2, _TILES, _CH, _K)
    zeros = jnp.zeros((_N, 128), jnp.float32)
    batch3 = batch.astype(jnp.int32).reshape(_N // 1000, 1, 1000)
    agg = _sc_agg_call()

    a1 = agg(jnp.concatenate([x, x_s], axis=0), ei, zeros)
    h1f_lo, h1f_hi = _layer1_one(x, a1[0], params["ef"][0])
    a2f = agg(jnp.concatenate([h1f_lo, h1f_hi], axis=0), ei, zeros)
    h1s_lo, h1s_hi = _layer1_one(x_s, a1[1], params["es"][0])
    a2s = agg(jnp.concatenate([h1s_lo, h1s_hi], axis=0), ei, zeros)
    n_f, g_f_raw = _tail_one(batch3, h1f_lo, h1f_hi, a2f[0], a2f[1],
                             params["ef"][1], params["pfn"])
    n_s, g_s_raw = _tail_one(batch3, h1s_lo, h1s_hi, a2s[0], a2s[1],
                             params["es"][1], params["psn"])
    b, g_f, g_s = _gheads(g_f_raw, g_s_raw,
                          params["pb"], params["pfg"], params["psg"])
    return (b, g_f, g_s, n_f, n_s)


# speedup vs baseline: 7.2796x; 1.0412x over previous
import functools

import jax
import jax.numpy as jnp
from jax import lax
from jax.experimental import pallas as pl
from jax.experimental.pallas import tpu as pltpu
from jax.experimental.pallas import tpu_sc as plsc

_N = 10000
_E = 320000
_G = 128
_TILES = 16
_K = 125
_CH = _E // _TILES // _K
_GC = 32
_NG = _CH // _GC
_RPT = 624
_RREM = _N - _TILES * _RPT


def _sc_agg_body(tab, ei, zeros, out, src_v, dst_v, rows_a, rows_b,
                 agg_sh, sem_a, sem_b, sem_i, sem_sa, sem_sb):
    c = lax.axis_index("c")
    s = lax.axis_index("s")
    r0 = s * _RPT
    tab_c = tab.at[pl.ds(c * _N, _N)]
    pltpu.sync_copy(zeros.at[pl.ds(r0, _RPT)], agg_sh.at[pl.ds(r0, _RPT)])

    @pl.when(s == 0)
    def _():
        pltpu.sync_copy(zeros.at[pl.ds(_TILES * _RPT, _RREM)],
                        agg_sh.at[pl.ds(_TILES * _RPT, _RREM)])

    pltpu.sync_copy(ei.at[0, s, pl.ds(0, _GC)], src_v.at[0])
    pltpu.sync_copy(ei.at[1, s, pl.ds(0, _GC)], dst_v.at[0])
    plsc.subcore_barrier()
    pltpu.async_copy(ei.at[0, s, pl.ds(_GC, _GC)], src_v.at[1], sem_i)
    pltpu.async_copy(ei.at[1, s, pl.ds(_GC, _GC)], dst_v.at[1], sem_i)
    pltpu.async_copy(tab_c.at[src_v.at[0, 0]], rows_a, sem_a)
    pltpu.async_copy(tab_c.at[src_v.at[0, 1]], rows_b, sem_b)

    @pl.loop(0, _CH // 2)
    def _pair(p):
        ch0 = 2 * p
        ch1 = ch0 + 1
        ch2 = ch0 + 2
        ch3 = ch0 + 3
        s0 = (ch0 // _GC) % 2
        s2 = (ch2 // _GC) % 2

        pltpu.make_async_copy(tab_c.at[src_v.at[s0, ch0 % _GC]],
                              rows_a, sem_a).wait()
        pltpu.async_copy(rows_a, agg_sh.at[dst_v.at[s0, ch0 % _GC]],
                         sem_sa, add=True)

        @pl.when(jnp.logical_and(ch2 % _GC == 0, ch2 < _CH))
        def _():
            pltpu.make_async_copy(ei.at[0, s, pl.ds(0, _GC)],
                                  src_v.at[0], sem_i).wait()
            pltpu.make_async_copy(ei.at[1, s, pl.ds(0, _GC)],
                                  dst_v.at[0], sem_i).wait()

        @pl.when(ch2 < _CH)
        def _():
            pltpu.make_async_copy(rows_a,
                                  agg_sh.at[dst_v.at[s0, ch0 % _GC]],
                                  sem_sa).wait()
            pltpu.async_copy(tab_c.at[src_v.at[s2, ch2 % _GC]], rows_a, sem_a)

        pltpu.make_async_copy(tab_c.at[src_v.at[s0, ch1 % _GC]],
                              rows_b, sem_b).wait()
        pltpu.async_copy(rows_b, agg_sh.at[dst_v.at[s0, ch1 % _GC]],
                         sem_sb, add=True)

        @pl.when(ch3 < _CH)
        def _():
            pltpu.make_async_copy(rows_b,
                                  agg_sh.at[dst_v.at[s0, ch1 % _GC]],
                                  sem_sb).wait()
            pltpu.async_copy(tab_c.at[src_v.at[s2, ch3 % _GC]], rows_b, sem_b)

        @pl.when(jnp.logical_and(ch2 % _GC == 0, ch2 + _GC < _CH))
        def _():
            g3 = ch2 // _GC + 1
            pltpu.async_copy(ei.at[0, s, pl.ds(g3 * _GC, _GC)],
                             src_v.at[g3 % 2], sem_i)
            pltpu.async_copy(ei.at[1, s, pl.ds(g3 * _GC, _GC)],
                             dst_v.at[g3 % 2], sem_i)

    pltpu.make_async_copy(
        rows_a, agg_sh.at[dst_v.at[((_CH - 2) // _GC) % 2, (_CH - 2) % _GC]],
        sem_sa).wait()
    pltpu.make_async_copy(
        rows_b, agg_sh.at[dst_v.at[((_CH - 1) // _GC) % 2, (_CH - 1) % _GC]],
        sem_sb).wait()
    plsc.subcore_barrier()

    pltpu.sync_copy(agg_sh.at[pl.ds(r0, _RPT)], out.at[c, pl.ds(r0, _RPT)])

    @pl.when(s == 0)
    def _():
        pltpu.sync_copy(agg_sh.at[pl.ds(_TILES * _RPT, _RREM)],
                        out.at[c, pl.ds(_TILES * _RPT, _RREM)])


@functools.lru_cache(maxsize=None)
def _sc_agg_call():
    mesh = plsc.VectorSubcoreMesh(core_axis_name="c", subcore_axis_name="s")
    return pl.kernel(
        _sc_agg_body,
        out_type=jax.ShapeDtypeStruct((2, _N, 128), jnp.float32),
        mesh=mesh,
        scratch_types=[
            pltpu.VMEM((2, _GC, _K), jnp.int32),
            pltpu.VMEM((2, _GC, _K), jnp.int32),
            pltpu.VMEM((_K, 128), jnp.float32),
            pltpu.VMEM((_K, 128), jnp.float32),
            pltpu.VMEM_SHARED((_N, 128), jnp.float32),
            pltpu.SemaphoreType.DMA,
            pltpu.SemaphoreType.DMA,
            pltpu.SemaphoreType.DMA,
            pltpu.SemaphoreType.DMA,
            pltpu.SemaphoreType.DMA,
        ],
    )


_BR = 1000


def _mlp_blk(u, w1, b1, w2, b2):
    t = jnp.maximum(
        jnp.dot(u, w1[...], preferred_element_type=jnp.float32) + b1[...], 0.0)
    return jnp.dot(t, w2[...], preferred_element_type=jnp.float32) + b2[...]


def _l1_body(x_ref, a_ref, w1, b1, w2, b2, out2):
    o = jnp.maximum(
        _mlp_blk(x_ref[...] + a_ref[0], w1, b1, w2, b2), 0.0)
    out2[0] = o[:, :128]
    out2[1] = o[:, 128:]


def _layer1_one(x, a1, enc, p):
    row = pl.BlockSpec((_BR, 128), lambda i: (i, 0))
    wspec = lambda shp: pl.BlockSpec(shp, lambda i: (0, 0))
    return pl.pallas_call(
        _l1_body,
        grid=(_N // _BR,),
        in_specs=[row,
                  pl.BlockSpec((1, _BR, 128), lambda i, e=enc: (e, i, 0)),
                  wspec((128, 256)), wspec((1, 256)),
                  wspec((256, 256)), wspec((1, 256))],
        out_specs=pl.BlockSpec((2, _BR, 128), lambda i: (0, i, 0)),
        out_shape=jax.ShapeDtypeStruct((2, _N, 128), jnp.float32),
    )(x, a1, p[0], p[1].reshape(1, -1), p[2], p[3].reshape(1, -1))


def _tail_body(batch_ref, h1_ref, a_ref,
               lw1, lb1, lw2, lb2, w1, b1, w2, b2, n_ref, g_ref):
    i = pl.program_id(0)
    u = jnp.concatenate(
        [h1_ref[0] + a_ref[0], h1_ref[1] + a_ref[1]], axis=1)
    h2 = jnp.maximum(_mlp_blk(u, lw1, lb1, lw2, lb2), 0.0)
    nc = jnp.concatenate([h1_ref[0], h1_ref[1], h2], axis=1)
    n_ref[...] = _mlp_blk(nc, w1, b1, w2, b2)

    bm = batch_ref[0]
    gi = lax.broadcasted_iota(jnp.int32, (_G, 1), 0)
    onehot = (gi == bm).astype(jnp.float32)

    @pl.when(i == 0)
    def _():
        g_ref[...] = jnp.zeros_like(g_ref)

    g_ref[...] += jnp.dot(onehot, nc, preferred_element_type=jnp.float32)


def _tail_one(batch3, h1, a2, pl2, p):
    pair = pl.BlockSpec((2, _BR, 128), lambda i: (0, i, 0))
    wspec = lambda shp: pl.BlockSpec(shp, lambda i: (0, 0))
    big = pl.BlockSpec((_BR, 512), lambda i: (i, 0))
    acc = pl.BlockSpec((_G, 512), lambda i: (0, 0))
    return pl.pallas_call(
        _tail_body,
        grid=(_N // _BR,),
        in_specs=[pl.BlockSpec((1, 1, _BR), lambda i: (i, 0, 0)),
                  pair, pair,
                  wspec((256, 256)), wspec((1, 256)),
                  wspec((256, 256)), wspec((1, 256)),
                  wspec((512, 512)), wspec((1, 512)),
                  wspec((512, 512)), wspec((1, 512))],
        out_specs=[big, acc],
        out_shape=[jax.ShapeDtypeStruct((_N, 512), jnp.float32),
                   jax.ShapeDtypeStruct((_G, 512), jnp.float32)],
    )(batch3, h1, a2,
      pl2[0], pl2[1].reshape(1, -1), pl2[2], pl2[3].reshape(1, -1),
      p[0], p[1].reshape(1, -1), p[2], p[3].reshape(1, -1))


def _gheads_body(gf_ref, gs_ref, bw1, bb1, bw2, bb2,
                 fw1, fb1, fw2, fb2, sw1, sb1, sw2, sb2,
                 b_ref, ogf_ref, ogs_ref):
    gcat = jnp.concatenate([gf_ref[...], gs_ref[...]], axis=1)
    b_ref[...] = _mlp_blk(gcat, bw1, bb1, bw2, bb2)
    ogf_ref[...] = _mlp_blk(gf_ref[...], fw1, fb1, fw2, fb2)
    ogs_ref[...] = _mlp_blk(gs_ref[...], sw1, sb1, sw2, sb2)


def _gheads(gf, gs, pb, pf, ps):
    wspec = lambda shp: pl.BlockSpec(shp, lambda i: (0, 0))
    g = pl.BlockSpec((_G, 512), lambda i: (0, 0))
    return pl.pallas_call(
        _gheads_body,
        grid=(1,),
        in_specs=[g, g,
                  wspec((1024, 512)), wspec((1, 512)),
                  wspec((512, 512)), wspec((1, 512)),
                  wspec((512, 512)), wspec((1, 512)),
                  wspec((512, 512)), wspec((1, 512)),
                  wspec((512, 512)), wspec((1, 512)),
                  wspec((512, 512)), wspec((1, 512))],
        out_specs=[g, g, g],
        out_shape=[jax.ShapeDtypeStruct((_G, 512), jnp.float32)] * 3,
    )(gf, gs,
      pb[0], pb[1].reshape(1, -1), pb[2], pb[3].reshape(1, -1),
      pf[0], pf[1].reshape(1, -1), pf[2], pf[3].reshape(1, -1),
      ps[0], ps[1].reshape(1, -1), ps[2], ps[3].reshape(1, -1))


def kernel(x, x_s, params, edge_index, batch):
    ei = edge_index.astype(jnp.int32).reshape(2, _TILES, _CH, _K)
    zeros = jnp.zeros((_N, 128), jnp.float32)
    batch3 = batch.astype(jnp.int32).reshape(_N // 1000, 1, 1000)
    agg = _sc_agg_call()

    a1 = agg(jnp.concatenate([x, x_s], axis=0), ei, zeros)
    h1f = _layer1_one(x, a1, 0, params["ef"][0])
    a2f = agg(h1f.reshape(2 * _N, 128), ei, zeros)
    h1s = _layer1_one(x_s, a1, 1, params["es"][0])
    a2s = agg(h1s.reshape(2 * _N, 128), ei, zeros)
    n_f, g_f_raw = _tail_one(batch3, h1f, a2f, params["ef"][1], params["pfn"])
    n_s, g_s_raw = _tail_one(batch3, h1s, a2s, params["es"][1], params["psn"])
    b, g_f, g_s = _gheads(g_f_raw, g_s_raw,
                          params["pb"], params["pfg"], params["psg"])
    return (b, g_f, g_s, n_f, n_s)
